# baseline (speedup 1.0000x reference)
import functools
import numpy as np
import jax
import jax.numpy as jnp
from jax.experimental import pallas as pl
from jax.experimental.pallas import tpu as pltpu

B, T, DIM = 1, 2048, 768
HEADS, DHEAD = 12, 64
HDIM = HEADS * DHEAD
CBS, SBS = 4, 4
NSEL, NMEM = 4, 1
WINDOW = 32
SCALE = 0.12
CDIM = CBS * DHEAD
HID = CDIM * 4
NBLK = T // CBS

QT = 256
NQT = T // QT
WT = 128
NWT = T // WT

NEG = -1e30


def _nt(a, b):
    return jax.lax.dot_general(a, b, (((1,), (1,)), ((), ())),
                               preferred_element_type=jnp.float32,
                               precision=jax.lax.Precision.HIGHEST)


def _nn(a, b):
    return jax.lax.dot_general(a, b, (((1,), (0,)), ((), ())),
                               preferred_element_type=jnp.float32,
                               precision=jax.lax.Precision.HIGHEST)


def _b(a):
    return a.astype(jnp.bfloat16)


def _ntd(a, b):
    return jax.lax.dot_general(_b(a), _b(b), (((1,), (1,)), ((), ())),
                               preferred_element_type=jnp.float32)


def _nnd(a, b):
    return jax.lax.dot_general(_b(a), _b(b), (((1,), (0,)), ((), ())),
                               preferred_element_type=jnp.float32)


def _rope_rot(x):
    n = x.shape[1]
    zl = pltpu.roll(x, n - 1, 1)
    zr = pltpu.roll(x, 1, 1)
    even = (jax.lax.broadcasted_iota(jnp.int32, x.shape, 1) % 2) == 0
    return jnp.where(even, -zl, zr)


def _qkv_kernel(x_ref, w_ref, c_ref, s_ref, q_ref, k_ref, v_ref):
    xt = x_ref[...]
    qkv = _ntd(xt, w_ref[...])
    q = qkv[:, :HDIM]
    k = qkv[:, HDIM:2 * HDIM]
    v = qkv[:, 2 * HDIM:]
    c = c_ref[...]
    s = s_ref[...]
    q_ref[...] = q * c + _rope_rot(q) * s
    k_ref[...] = k * c + _rope_rot(k) * s
    v_ref[...] = v


def _compress_kernel(km_ref, vm_ref, kp_ref, vp_ref,
                     kfc_ref, kpj_ref, vfc_ref, vpj_ref,
                     ck_ref, cv_ref):
    km = km_ref[0] + kp_ref[0]
    hk = _ntd(km, kfc_ref[...])
    hk = jnp.square(jnp.maximum(hk, 0.0))
    ck_ref[0] = _ntd(hk, kpj_ref[...])
    vm = vm_ref[0] + vp_ref[0]
    hv = _ntd(vm, vfc_ref[...])
    hv = jnp.square(jnp.maximum(hv, 0.0))
    cv_ref[0] = _ntd(hv, vpj_ref[...])


def _cattn_kernel(q_ref, ck_ref, cv_ref, mk_ref, mv_ref, cout_ref, imp_ref):
    i = pl.program_id(0)
    h = pl.program_id(1)
    q = q_ref[0]
    ck = ck_ref[0]
    sim = _ntd(q, ck) * SCALE
    t = i * QT + jax.lax.broadcasted_iota(jnp.int32, (QT, NBLK), 0)
    b = jax.lax.broadcasted_iota(jnp.int32, (QT, NBLK), 1)
    mask = (CBS * b + CBS - 1) < t
    sim = jnp.where(mask, sim, NEG)
    qb = _b(q).astype(jnp.float32)
    mkb = _b(mk_ref[0]).astype(jnp.float32)
    mem_logit = jnp.sum(qb * mkb, axis=-1, keepdims=True) * SCALE
    m = jnp.maximum(jnp.max(sim, axis=-1, keepdims=True), mem_logit)
    p = jnp.exp(sim - m)
    pm = jnp.exp(mem_logit - m)
    denom = jnp.sum(p, axis=-1, keepdims=True) + pm
    attn_n = p / denom
    pm_n = pm / denom
    memo = (_b(pm_n).astype(jnp.float32)) * (_b(mv_ref[0]).astype(jnp.float32))
    cout_ref[0] = _nnd(attn_n, cv_ref[0]) + memo

    @pl.when(h == 0)
    def _():
        imp_ref[...] = attn_n

    @pl.when(h > 0)
    def _():
        imp_ref[...] += attn_n


KCH = 512
NCH = T // KCH


def _fine_kernel(imp_ref, q_ref, k_ref, v_ref, fout_ref,
                 sel_ref, m_ref, l_ref, acc_ref):
    i = pl.program_id(0)
    h = pl.program_id(1)
    j = pl.program_id(2)

    @pl.when((h == 0) & (j == 0))
    def _():
        imp = imp_ref[...]
        cols = jax.lax.broadcasted_iota(jnp.int32, (QT, NBLK), 1)
        cur = imp
        for n in range(NSEL):
            m = jnp.max(cur, axis=-1, keepdims=True)
            idx = jnp.min(jnp.where(cur == m, cols, NBLK),
                          axis=-1, keepdims=True)
            sel_ref[:, n:n + 1] = idx.astype(jnp.float32)
            cur = jnp.where(cols == idx, -1.0, cur)

    @pl.when(j == 0)
    def _():
        m_ref[:, 0:1] = jnp.full((QT, 1), NEG, jnp.float32)
        l_ref[:, 0:1] = jnp.zeros((QT, 1), jnp.float32)
        acc_ref[...] = jnp.zeros((QT, DHEAD), jnp.float32)

    @pl.when(2 * j <= i)
    def _():
        q = q_ref[0]
        k = k_ref[0]
        sim = _ntd(q, k) * SCALE
        t = i * QT + jax.lax.broadcasted_iota(jnp.int32, (QT, KCH), 0)
        s = j * KCH + jax.lax.broadcasted_iota(jnp.int32, (QT, KCH), 1)
        sblk = s // SBS
        mask = sblk == (t // SBS)
        for n in range(NSEL):
            seln = sel_ref[:, n:n + 1].astype(jnp.int32)
            mask = mask | (sblk == seln)
        mask = mask & (s <= t)
        sim = jnp.where(mask, sim, NEG)
        mprev = m_ref[:, 0:1]
        mnew = jnp.maximum(mprev, jnp.max(sim, axis=-1, keepdims=True))
        p = jnp.where(mask, jnp.exp(sim - mnew), 0.0)
        corr = jnp.exp(mprev - mnew)
        m_ref[:, 0:1] = mnew
        l_ref[:, 0:1] = l_ref[:, 0:1] * corr + jnp.sum(p, axis=-1,
                                                       keepdims=True)
        acc_ref[...] = acc_ref[...] * corr + _nnd(p, v_ref[0])

    @pl.when(j == NCH - 1)
    def _():
        fout_ref[0] = acc_ref[...] / l_ref[:, 0:1]


def _window_kernel(q_ref, kp_ref, kc_ref, vp_ref, vc_ref, sout_ref):
    i = pl.program_id(0)
    q = q_ref[0]
    t = i * WT + jax.lax.broadcasted_iota(jnp.int32, (WT, WT), 0)
    scol = jax.lax.broadcasted_iota(jnp.int32, (WT, WT), 1)

    sp = _ntd(q, kp_ref[0]) * SCALE
    pos_p = (i - 1) * WT + scol
    mask_p = (pos_p >= 0) & (t - pos_p < WINDOW) & (pos_p <= t)
    sp = jnp.where(mask_p, sp, NEG)

    sc = _ntd(q, kc_ref[0]) * SCALE
    pos_c = i * WT + scol
    mask_c = (pos_c <= t) & (t - pos_c < WINDOW)
    sc = jnp.where(mask_c, sc, NEG)

    m = jnp.maximum(jnp.max(sp, axis=-1, keepdims=True),
                    jnp.max(sc, axis=-1, keepdims=True))
    pp = jnp.exp(sp - m)
    pc = jnp.exp(sc - m)
    denom = (jnp.sum(pp, axis=-1, keepdims=True)
             + jnp.sum(pc, axis=-1, keepdims=True))
    sout_ref[0] = _nnd(pp / denom, vp_ref[0]) + _nnd(pc / denom, vc_ref[0])


def _combine_kernel(x_ref, co_ref, fo_ref, so_ref,
                    sw_ref, sb_ref, cwh_ref, out_ref):
    xt = x_ref[...]
    glog = _ntd(xt, sw_ref[...]) + sb_ref[...]
    g = 1.0 / (1.0 + jnp.exp(-glog))
    acc = jnp.zeros((QT, DIM), jnp.float32)
    for h in range(HEADS):
        oh = (g[:, 3 * h:3 * h + 1] * co_ref[h]
              + g[:, 3 * h + 1:3 * h + 2] * fo_ref[h]
              + g[:, 3 * h + 2:3 * h + 3] * so_ref[h])
        acc = acc + _ntd(oh, cwh_ref[h])
    out_ref[...] = acc


def _rope_tables():
    inv = 1.0 / (10000.0 ** (jnp.arange(0, DHEAD, 2, dtype=jnp.float32) / DHEAD))
    freqs = jnp.arange(T, dtype=jnp.float32)[:, None] * inv[None, :]
    cos = jnp.repeat(jnp.cos(freqs), 2, axis=1)
    sin = jnp.repeat(jnp.sin(freqs), 2, axis=1)
    return jnp.tile(cos, (1, HEADS)), jnp.tile(sin, (1, HEADS))


@jax.jit
def kernel(x, qkv_w, k_fc_w, k_proj_w, v_fc_w, v_proj_w, compress_mem_kv,
           k_pos, v_pos, strat_w, strat_b, combine_w):
    f32 = jnp.float32
    x2 = x[0]
    wq = qkv_w.reshape(3 * HDIM, DIM)
    cos, sin = _rope_tables()

    q, k, v = pl.pallas_call(
        _qkv_kernel,
        grid=(NQT,),
        in_specs=[
            pl.BlockSpec((QT, DIM), lambda i: (i, 0)),
            pl.BlockSpec((3 * HDIM, DIM), lambda i: (0, 0)),
            pl.BlockSpec((QT, HDIM), lambda i: (i, 0)),
            pl.BlockSpec((QT, HDIM), lambda i: (i, 0)),
        ],
        out_specs=[
            pl.BlockSpec((QT, HDIM), lambda i: (i, 0)),
            pl.BlockSpec((QT, HDIM), lambda i: (i, 0)),
            pl.BlockSpec((QT, HDIM), lambda i: (i, 0)),
        ],
        out_shape=[jax.ShapeDtypeStruct((T, HDIM), f32)] * 3,
    )(x2, wq, cos, sin)

    qh = q.reshape(T, HEADS, DHEAD).transpose(1, 0, 2)
    kh = k.reshape(T, HEADS, DHEAD).transpose(1, 0, 2)
    vh = v.reshape(T, HEADS, DHEAD).transpose(1, 0, 2)
    km = kh.reshape(HEADS, NBLK, CDIM)
    vm = vh.reshape(HEADS, NBLK, CDIM)
    kp = k_pos.reshape(HEADS, 1, CDIM)
    vp = v_pos.reshape(HEADS, 1, CDIM)

    ck, cv = pl.pallas_call(
        _compress_kernel,
        grid=(HEADS,),
        in_specs=[
            pl.BlockSpec((1, NBLK, CDIM), lambda h: (h, 0, 0)),
            pl.BlockSpec((1, NBLK, CDIM), lambda h: (h, 0, 0)),
            pl.BlockSpec((1, 1, CDIM), lambda h: (h, 0, 0)),
            pl.BlockSpec((1, 1, CDIM), lambda h: (h, 0, 0)),
            pl.BlockSpec((HID, CDIM), lambda h: (0, 0)),
            pl.BlockSpec((DHEAD, HID), lambda h: (0, 0)),
            pl.BlockSpec((HID, CDIM), lambda h: (0, 0)),
            pl.BlockSpec((DHEAD, HID), lambda h: (0, 0)),
        ],
        out_specs=[
            pl.BlockSpec((1, NBLK, DHEAD), lambda h: (h, 0, 0)),
            pl.BlockSpec((1, NBLK, DHEAD), lambda h: (h, 0, 0)),
        ],
        out_shape=[jax.ShapeDtypeStruct((HEADS, NBLK, DHEAD), f32)] * 2,
    )(km, vm, kp, vp, k_fc_w, k_proj_w, v_fc_w, v_proj_w)

    mem_k = compress_mem_kv[0].reshape(HEADS, NMEM, DHEAD)
    mem_v = compress_mem_kv[1].reshape(HEADS, NMEM, DHEAD)

    cout, imp = pl.pallas_call(
        _cattn_kernel,
        grid=(NQT, HEADS),
        in_specs=[
            pl.BlockSpec((1, QT, DHEAD), lambda i, h: (h, i, 0)),
            pl.BlockSpec((1, NBLK, DHEAD), lambda i, h: (h, 0, 0)),
            pl.BlockSpec((1, NBLK, DHEAD), lambda i, h: (h, 0, 0)),
            pl.BlockSpec((1, NMEM, DHEAD), lambda i, h: (h, 0, 0)),
            pl.BlockSpec((1, NMEM, DHEAD), lambda i, h: (h, 0, 0)),
        ],
        out_specs=[
            pl.BlockSpec((1, QT, DHEAD), lambda i, h: (h, i, 0)),
            pl.BlockSpec((QT, NBLK), lambda i, h: (i, 0)),
        ],
        out_shape=[
            jax.ShapeDtypeStruct((HEADS, T, DHEAD), f32),
            jax.ShapeDtypeStruct((T, NBLK), f32),
        ],
    )(qh, ck, cv, mem_k, mem_v)

    fout = pl.pallas_call(
        _fine_kernel,
        grid=(NQT, HEADS, NCH),
        in_specs=[
            pl.BlockSpec((QT, NBLK), lambda i, h, j: (i, 0)),
            pl.BlockSpec((1, QT, DHEAD), lambda i, h, j: (h, i, 0)),
            pl.BlockSpec((1, KCH, DHEAD),
                         lambda i, h, j: (h, jnp.minimum(j, i // 2), 0)),
            pl.BlockSpec((1, KCH, DHEAD),
                         lambda i, h, j: (h, jnp.minimum(j, i // 2), 0)),
        ],
        out_specs=pl.BlockSpec((1, QT, DHEAD), lambda i, h, j: (h, i, 0)),
        out_shape=jax.ShapeDtypeStruct((HEADS, T, DHEAD), f32),
        scratch_shapes=[pltpu.VMEM((QT, 128), f32),
                        pltpu.VMEM((QT, 128), f32),
                        pltpu.VMEM((QT, 128), f32),
                        pltpu.VMEM((QT, DHEAD), f32)],
    )(imp, qh, kh, vh)

    sout = pl.pallas_call(
        _window_kernel,
        grid=(NWT, HEADS),
        in_specs=[
            pl.BlockSpec((1, WT, DHEAD), lambda i, h: (h, i, 0)),
            pl.BlockSpec((1, WT, DHEAD),
                         lambda i, h: (h, jnp.maximum(i - 1, 0), 0)),
            pl.BlockSpec((1, WT, DHEAD), lambda i, h: (h, i, 0)),
            pl.BlockSpec((1, WT, DHEAD),
                         lambda i, h: (h, jnp.maximum(i - 1, 0), 0)),
            pl.BlockSpec((1, WT, DHEAD), lambda i, h: (h, i, 0)),
        ],
        out_specs=pl.BlockSpec((1, WT, DHEAD), lambda i, h: (h, i, 0)),
        out_shape=jax.ShapeDtypeStruct((HEADS, T, DHEAD), f32),
    )(qh, kh, kh, vh, vh)

    sw = jnp.zeros((128, DIM), f32).at[:3 * HEADS].set(strat_w)
    sb = jnp.zeros((1, 128), f32).at[0, :3 * HEADS].set(strat_b)
    cwh = combine_w.reshape(DIM, HEADS, DHEAD).transpose(1, 0, 2)
    out = pl.pallas_call(
        _combine_kernel,
        grid=(NQT,),
        in_specs=[
            pl.BlockSpec((QT, DIM), lambda i: (i, 0)),
            pl.BlockSpec((HEADS, QT, DHEAD), lambda i: (0, i, 0)),
            pl.BlockSpec((HEADS, QT, DHEAD), lambda i: (0, i, 0)),
            pl.BlockSpec((HEADS, QT, DHEAD), lambda i: (0, i, 0)),
            pl.BlockSpec((128, DIM), lambda i: (0, 0)),
            pl.BlockSpec((1, 128), lambda i: (0, 0)),
            pl.BlockSpec((HEADS, DIM, DHEAD), lambda i: (0, 0, 0)),
        ],
        out_specs=pl.BlockSpec((QT, DIM), lambda i: (i, 0)),
        out_shape=jax.ShapeDtypeStruct((T, DIM), f32),
    )(x2, cout, fout, sout, sw, sb, cwh)

    return out[None]


# speedup vs baseline: 1.4548x; 1.4548x over previous
import functools
import numpy as np
import jax
import jax.numpy as jnp
from jax.experimental import pallas as pl
from jax.experimental.pallas import tpu as pltpu

B, T, DIM = 1, 2048, 768
HEADS, DHEAD = 12, 64
HDIM = HEADS * DHEAD
CBS, SBS = 4, 4
NSEL, NMEM = 4, 1
WINDOW = 32
SCALE = 0.12
CDIM = CBS * DHEAD
HID = CDIM * 4
NBLK = T // CBS

QT = 256
NQT = T // QT
WT = 128
NWT = T // WT

NEG = -1e30


def _nt(a, b):
    return jax.lax.dot_general(a, b, (((1,), (1,)), ((), ())),
                               preferred_element_type=jnp.float32,
                               precision=jax.lax.Precision.HIGHEST)


def _nn(a, b):
    return jax.lax.dot_general(a, b, (((1,), (0,)), ((), ())),
                               preferred_element_type=jnp.float32,
                               precision=jax.lax.Precision.HIGHEST)


def _b(a):
    return a.astype(jnp.bfloat16)


def _ntd(a, b):
    return jax.lax.dot_general(_b(a), _b(b), (((1,), (1,)), ((), ())),
                               preferred_element_type=jnp.float32)


def _nnd(a, b):
    return jax.lax.dot_general(_b(a), _b(b), (((1,), (0,)), ((), ())),
                               preferred_element_type=jnp.float32)


def _rope_rot(x):
    n = x.shape[1]
    zl = pltpu.roll(x, n - 1, 1)
    zr = pltpu.roll(x, 1, 1)
    even = (jax.lax.broadcasted_iota(jnp.int32, x.shape, 1) % 2) == 0
    return jnp.where(even, -zl, zr)


def _qkv_kernel(x_ref, w_ref, c_ref, s_ref, q_ref, k_ref, v_ref):
    xt = x_ref[...]
    qkv = _ntd(xt, w_ref[...])
    q = qkv[:, :HDIM]
    k = qkv[:, HDIM:2 * HDIM]
    v = qkv[:, 2 * HDIM:]
    c = c_ref[...]
    s = s_ref[...]
    qr = q * c + _rope_rot(q) * s
    kr = k * c + _rope_rot(k) * s
    for h in range(HEADS):
        q_ref[h] = qr[:, h * DHEAD:(h + 1) * DHEAD]
        k_ref[h] = kr[:, h * DHEAD:(h + 1) * DHEAD]
        v_ref[h] = v[:, h * DHEAD:(h + 1) * DHEAD]


def _compress_kernel(km_ref, vm_ref, kp_ref, vp_ref,
                     kfc_ref, kpj_ref, vfc_ref, vpj_ref,
                     ck_ref, cv_ref):
    km = km_ref[0] + kp_ref[0]
    hk = _ntd(km, kfc_ref[...])
    hk = jnp.square(jnp.maximum(hk, 0.0))
    ck_ref[0] = _ntd(hk, kpj_ref[...])
    vm = vm_ref[0] + vp_ref[0]
    hv = _ntd(vm, vfc_ref[...])
    hv = jnp.square(jnp.maximum(hv, 0.0))
    cv_ref[0] = _ntd(hv, vpj_ref[...])


def _cattn_kernel(q_ref, ck_ref, cv_ref, mk_ref, mv_ref, cout_ref, imp_ref,
                  bias_ref):
    i = pl.program_id(0)
    h = pl.program_id(1)

    @pl.when(h == 0)
    def _():
        t = i * QT + jax.lax.broadcasted_iota(jnp.int32, (QT, NBLK), 0)
        b = jax.lax.broadcasted_iota(jnp.int32, (QT, NBLK), 1)
        mask = (CBS * b + CBS - 1) < t
        bias_ref[...] = jnp.where(mask, 0.0, NEG)

    q = q_ref[0]
    ck = ck_ref[0]
    sim = _ntd(q, ck) * SCALE + bias_ref[...]
    qb = _b(q).astype(jnp.float32)
    mkb = _b(mk_ref[0]).astype(jnp.float32)
    mem_logit = jnp.sum(qb * mkb, axis=-1, keepdims=True) * SCALE
    m = jnp.maximum(jnp.max(sim, axis=-1, keepdims=True), mem_logit)
    p = jnp.exp(sim - m)
    pm = jnp.exp(mem_logit - m)
    denom = jnp.sum(p, axis=-1, keepdims=True) + pm
    attn_n = p / denom
    pm_n = pm / denom
    memo = (_b(pm_n).astype(jnp.float32)) * (_b(mv_ref[0]).astype(jnp.float32))
    cout_ref[0] = _nnd(attn_n, cv_ref[0]) + memo

    @pl.when(h == 0)
    def _():
        imp_ref[...] = attn_n

    @pl.when(h > 0)
    def _():
        imp_ref[...] += attn_n


def _fine_kernel(imp_ref, q_ref, k_ref, v_ref, fout_ref, bias_ref):
    i = pl.program_id(0)
    h = pl.program_id(1)

    @pl.when(h == 0)
    def _():
        imp = imp_ref[...]
        cols = jax.lax.broadcasted_iota(jnp.int32, (QT, NBLK), 1)
        cur = imp
        sels = []
        for n in range(NSEL):
            m = jnp.max(cur, axis=-1, keepdims=True)
            idx = jnp.min(jnp.where(cur == m, cols, NBLK),
                          axis=-1, keepdims=True)
            sels.append(idx)
            cur = jnp.where(cols == idx, -1.0, cur)
        t = i * QT + jax.lax.broadcasted_iota(jnp.int32, (QT, T), 0)
        s = jax.lax.broadcasted_iota(jnp.int32, (QT, T), 1)
        sblk = s // SBS
        mask = sblk == (t // SBS)
        for n in range(NSEL):
            mask = mask | (sblk == sels[n])
        mask = mask & (s <= t)
        bias_ref[...] = jnp.where(mask, 0.0, NEG)

    q = q_ref[0]
    k = k_ref[0]
    sim = _ntd(q, k) * SCALE + bias_ref[...]
    m = jnp.max(sim, axis=-1, keepdims=True)
    p = jnp.exp(sim - m)
    denom = jnp.sum(p, axis=-1, keepdims=True)
    fout_ref[0] = _nnd(p / denom, v_ref[0])


def _window_kernel(q_ref, kp_ref, kc_ref, vp_ref, vc_ref, sout_ref):
    i = pl.program_id(0)
    q = q_ref[0]
    t = i * WT + jax.lax.broadcasted_iota(jnp.int32, (WT, WT), 0)
    scol = jax.lax.broadcasted_iota(jnp.int32, (WT, WT), 1)

    sp = _ntd(q, kp_ref[0]) * SCALE
    pos_p = (i - 1) * WT + scol
    mask_p = (pos_p >= 0) & (t - pos_p < WINDOW) & (pos_p <= t)
    sp = jnp.where(mask_p, sp, NEG)

    sc = _ntd(q, kc_ref[0]) * SCALE
    pos_c = i * WT + scol
    mask_c = (pos_c <= t) & (t - pos_c < WINDOW)
    sc = jnp.where(mask_c, sc, NEG)

    m = jnp.maximum(jnp.max(sp, axis=-1, keepdims=True),
                    jnp.max(sc, axis=-1, keepdims=True))
    pp = jnp.exp(sp - m)
    pc = jnp.exp(sc - m)
    denom = (jnp.sum(pp, axis=-1, keepdims=True)
             + jnp.sum(pc, axis=-1, keepdims=True))
    sout_ref[0] = _nnd(pp / denom, vp_ref[0]) + _nnd(pc / denom, vc_ref[0])


def _combine_kernel(x_ref, co_ref, fo_ref, so_ref,
                    sw_ref, sb_ref, cwh_ref, out_ref):
    xt = x_ref[...]
    glog = _ntd(xt, sw_ref[...]) + sb_ref[...]
    g = 1.0 / (1.0 + jnp.exp(-glog))
    acc = jnp.zeros((QT, DIM), jnp.float32)
    for h in range(HEADS):
        oh = (g[:, 3 * h:3 * h + 1] * co_ref[h]
              + g[:, 3 * h + 1:3 * h + 2] * fo_ref[h]
              + g[:, 3 * h + 2:3 * h + 3] * so_ref[h])
        acc = acc + _ntd(oh, cwh_ref[h])
    out_ref[...] = acc


def _rope_tables():
    inv = 1.0 / (10000.0 ** (jnp.arange(0, DHEAD, 2, dtype=jnp.float32) / DHEAD))
    freqs = jnp.arange(T, dtype=jnp.float32)[:, None] * inv[None, :]
    cos = jnp.repeat(jnp.cos(freqs), 2, axis=1)
    sin = jnp.repeat(jnp.sin(freqs), 2, axis=1)
    return jnp.tile(cos, (1, HEADS)), jnp.tile(sin, (1, HEADS))


@jax.jit
def kernel(x, qkv_w, k_fc_w, k_proj_w, v_fc_w, v_proj_w, compress_mem_kv,
           k_pos, v_pos, strat_w, strat_b, combine_w):
    f32 = jnp.float32
    x2 = x[0]
    wq = qkv_w.reshape(3 * HDIM, DIM)
    cos, sin = _rope_tables()

    qh, kh, vh = pl.pallas_call(
        _qkv_kernel,
        grid=(NQT,),
        in_specs=[
            pl.BlockSpec((QT, DIM), lambda i: (i, 0)),
            pl.BlockSpec((3 * HDIM, DIM), lambda i: (0, 0)),
            pl.BlockSpec((QT, HDIM), lambda i: (i, 0)),
            pl.BlockSpec((QT, HDIM), lambda i: (i, 0)),
        ],
        out_specs=[
            pl.BlockSpec((HEADS, QT, DHEAD), lambda i: (0, i, 0)),
            pl.BlockSpec((HEADS, QT, DHEAD), lambda i: (0, i, 0)),
            pl.BlockSpec((HEADS, QT, DHEAD), lambda i: (0, i, 0)),
        ],
        out_shape=[jax.ShapeDtypeStruct((HEADS, T, DHEAD), f32)] * 3,
    )(x2, wq, cos, sin)

    km = kh.reshape(HEADS, NBLK, CDIM)
    vm = vh.reshape(HEADS, NBLK, CDIM)
    kp = k_pos.reshape(HEADS, 1, CDIM)
    vp = v_pos.reshape(HEADS, 1, CDIM)

    ck, cv = pl.pallas_call(
        _compress_kernel,
        grid=(HEADS,),
        in_specs=[
            pl.BlockSpec((1, NBLK, CDIM), lambda h: (h, 0, 0)),
            pl.BlockSpec((1, NBLK, CDIM), lambda h: (h, 0, 0)),
            pl.BlockSpec((1, 1, CDIM), lambda h: (h, 0, 0)),
            pl.BlockSpec((1, 1, CDIM), lambda h: (h, 0, 0)),
            pl.BlockSpec((HID, CDIM), lambda h: (0, 0)),
            pl.BlockSpec((DHEAD, HID), lambda h: (0, 0)),
            pl.BlockSpec((HID, CDIM), lambda h: (0, 0)),
            pl.BlockSpec((DHEAD, HID), lambda h: (0, 0)),
        ],
        out_specs=[
            pl.BlockSpec((1, NBLK, DHEAD), lambda h: (h, 0, 0)),
            pl.BlockSpec((1, NBLK, DHEAD), lambda h: (h, 0, 0)),
        ],
        out_shape=[jax.ShapeDtypeStruct((HEADS, NBLK, DHEAD), f32)] * 2,
    )(km, vm, kp, vp, k_fc_w, k_proj_w, v_fc_w, v_proj_w)

    mem_k = compress_mem_kv[0].reshape(HEADS, NMEM, DHEAD)
    mem_v = compress_mem_kv[1].reshape(HEADS, NMEM, DHEAD)

    cout, imp = pl.pallas_call(
        _cattn_kernel,
        grid=(NQT, HEADS),
        in_specs=[
            pl.BlockSpec((1, QT, DHEAD), lambda i, h: (h, i, 0)),
            pl.BlockSpec((1, NBLK, DHEAD), lambda i, h: (h, 0, 0)),
            pl.BlockSpec((1, NBLK, DHEAD), lambda i, h: (h, 0, 0)),
            pl.BlockSpec((1, NMEM, DHEAD), lambda i, h: (h, 0, 0)),
            pl.BlockSpec((1, NMEM, DHEAD), lambda i, h: (h, 0, 0)),
        ],
        out_specs=[
            pl.BlockSpec((1, QT, DHEAD), lambda i, h: (h, i, 0)),
            pl.BlockSpec((QT, NBLK), lambda i, h: (i, 0)),
        ],
        out_shape=[
            jax.ShapeDtypeStruct((HEADS, T, DHEAD), f32),
            jax.ShapeDtypeStruct((T, NBLK), f32),
        ],
        scratch_shapes=[pltpu.VMEM((QT, NBLK), f32)],
    )(qh, ck, cv, mem_k, mem_v)

    fout = pl.pallas_call(
        _fine_kernel,
        grid=(NQT, HEADS),
        in_specs=[
            pl.BlockSpec((QT, NBLK), lambda i, h: (i, 0)),
            pl.BlockSpec((1, QT, DHEAD), lambda i, h: (h, i, 0)),
            pl.BlockSpec((1, T, DHEAD), lambda i, h: (h, 0, 0)),
            pl.BlockSpec((1, T, DHEAD), lambda i, h: (h, 0, 0)),
        ],
        out_specs=pl.BlockSpec((1, QT, DHEAD), lambda i, h: (h, i, 0)),
        out_shape=jax.ShapeDtypeStruct((HEADS, T, DHEAD), f32),
        scratch_shapes=[pltpu.VMEM((QT, T), f32)],
    )(imp, qh, kh, vh)

    sout = pl.pallas_call(
        _window_kernel,
        grid=(NWT, HEADS),
        in_specs=[
            pl.BlockSpec((1, WT, DHEAD), lambda i, h: (h, i, 0)),
            pl.BlockSpec((1, WT, DHEAD),
                         lambda i, h: (h, jnp.maximum(i - 1, 0), 0)),
            pl.BlockSpec((1, WT, DHEAD), lambda i, h: (h, i, 0)),
            pl.BlockSpec((1, WT, DHEAD),
                         lambda i, h: (h, jnp.maximum(i - 1, 0), 0)),
            pl.BlockSpec((1, WT, DHEAD), lambda i, h: (h, i, 0)),
        ],
        out_specs=pl.BlockSpec((1, WT, DHEAD), lambda i, h: (h, i, 0)),
        out_shape=jax.ShapeDtypeStruct((HEADS, T, DHEAD), f32),
    )(qh, kh, kh, vh, vh)

    sw = jnp.zeros((128, DIM), f32).at[:3 * HEADS].set(strat_w)
    sb = jnp.zeros((1, 128), f32).at[0, :3 * HEADS].set(strat_b)
    cwh = combine_w.reshape(DIM, HEADS, DHEAD).transpose(1, 0, 2)
    out = pl.pallas_call(
        _combine_kernel,
        grid=(NQT,),
        in_specs=[
            pl.BlockSpec((QT, DIM), lambda i: (i, 0)),
            pl.BlockSpec((HEADS, QT, DHEAD), lambda i: (0, i, 0)),
            pl.BlockSpec((HEADS, QT, DHEAD), lambda i: (0, i, 0)),
            pl.BlockSpec((HEADS, QT, DHEAD), lambda i: (0, i, 0)),
            pl.BlockSpec((128, DIM), lambda i: (0, 0)),
            pl.BlockSpec((1, 128), lambda i: (0, 0)),
            pl.BlockSpec((HEADS, DIM, DHEAD), lambda i: (0, 0, 0)),
        ],
        out_specs=pl.BlockSpec((QT, DIM), lambda i: (i, 0)),
        out_shape=jax.ShapeDtypeStruct((T, DIM), f32),
    )(x2, cout, fout, sout, sw, sb, cwh)

    return out[None]


# speedup vs baseline: 1.5411x; 1.0593x over previous
import functools
import numpy as np
import jax
import jax.numpy as jnp
from jax.experimental import pallas as pl
from jax.experimental.pallas import tpu as pltpu

B, T, DIM = 1, 2048, 768
HEADS, DHEAD = 12, 64
HDIM = HEADS * DHEAD
CBS, SBS = 4, 4
NSEL, NMEM = 4, 1
WINDOW = 32
SCALE = 0.12
CDIM = CBS * DHEAD
HID = CDIM * 4
NBLK = T // CBS

QT = 256
NQT = T // QT
WT = 128
NWT = T // WT

NEG = -1e30


def _nt(a, b):
    return jax.lax.dot_general(a, b, (((1,), (1,)), ((), ())),
                               preferred_element_type=jnp.float32,
                               precision=jax.lax.Precision.HIGHEST)


def _nn(a, b):
    return jax.lax.dot_general(a, b, (((1,), (0,)), ((), ())),
                               preferred_element_type=jnp.float32,
                               precision=jax.lax.Precision.HIGHEST)


def _b(a):
    return a.astype(jnp.bfloat16)


def _ntd(a, b):
    return jax.lax.dot_general(_b(a), _b(b), (((1,), (1,)), ((), ())),
                               preferred_element_type=jnp.float32)


def _nnd(a, b):
    return jax.lax.dot_general(_b(a), _b(b), (((1,), (0,)), ((), ())),
                               preferred_element_type=jnp.float32)


def _rope_rot(x):
    n = x.shape[1]
    zl = pltpu.roll(x, n - 1, 1)
    zr = pltpu.roll(x, 1, 1)
    even = (jax.lax.broadcasted_iota(jnp.int32, x.shape, 1) % 2) == 0
    return jnp.where(even, -zl, zr)


def _qkv_kernel(x_ref, w_ref, c_ref, s_ref, q_ref, k_ref, v_ref):
    xt = x_ref[...]
    qkv = _ntd(xt, w_ref[...])
    q = qkv[:, :HDIM]
    k = qkv[:, HDIM:2 * HDIM]
    v = qkv[:, 2 * HDIM:]
    c = c_ref[...]
    s = s_ref[...]
    qr = q * c + _rope_rot(q) * s
    kr = k * c + _rope_rot(k) * s
    for h in range(HEADS):
        q_ref[h] = qr[:, h * DHEAD:(h + 1) * DHEAD]
        k_ref[h] = kr[:, h * DHEAD:(h + 1) * DHEAD]
        v_ref[h] = v[:, h * DHEAD:(h + 1) * DHEAD]


def _compress_kernel(km_ref, vm_ref, kp_ref, vp_ref,
                     kfc_ref, kpj_ref, vfc_ref, vpj_ref,
                     ck_ref, cv_ref):
    km = km_ref[0] + kp_ref[0]
    hk = _ntd(km, kfc_ref[...])
    hk = jnp.square(jnp.maximum(hk, 0.0))
    ck_ref[0] = _ntd(hk, kpj_ref[...])
    vm = vm_ref[0] + vp_ref[0]
    hv = _ntd(vm, vfc_ref[...])
    hv = jnp.square(jnp.maximum(hv, 0.0))
    cv_ref[0] = _ntd(hv, vpj_ref[...])


def _cattn_kernel(q_ref, ck_ref, cv_ref, mk_ref, mv_ref, cout_ref, imp_ref,
                  bias_ref):
    i = pl.program_id(0)
    h = pl.program_id(1)

    @pl.when(h == 0)
    def _():
        t = i * QT + jax.lax.broadcasted_iota(jnp.int32, (QT, NBLK), 0)
        b = jax.lax.broadcasted_iota(jnp.int32, (QT, NBLK), 1)
        mask = (CBS * b + CBS - 1) < t
        bias_ref[...] = jnp.where(mask, 0.0, NEG)

    q = q_ref[0]
    ck = ck_ref[0]
    sim = _ntd(q, ck) * SCALE + bias_ref[...]
    qb = _b(q).astype(jnp.float32)
    mkb = _b(mk_ref[0]).astype(jnp.float32)
    mem_logit = jnp.sum(qb * mkb, axis=-1, keepdims=True) * SCALE
    m = jnp.maximum(jnp.max(sim, axis=-1, keepdims=True), mem_logit)
    p = jnp.exp(sim - m)
    pm = jnp.exp(mem_logit - m)
    denom = jnp.sum(p, axis=-1, keepdims=True) + pm
    attn_n = p / denom
    pm_n = pm / denom
    memo = (_b(pm_n).astype(jnp.float32)) * (_b(mv_ref[0]).astype(jnp.float32))
    cout_ref[0] = _nnd(attn_n, cv_ref[0]) + memo

    @pl.when(h == 0)
    def _():
        imp_ref[...] = attn_n

    @pl.when(h > 0)
    def _():
        imp_ref[...] += attn_n


def _fine_kernel(imp_ref, q_ref, k_ref, v_ref, fout_ref, bias_ref):
    h = pl.program_id(0)
    i = pl.program_id(1)

    @pl.when(h == 0)
    def _():
        imp = imp_ref[...]
        cols = jax.lax.broadcasted_iota(jnp.int32, (QT, NBLK), 1)
        cur = imp
        sels = []
        for n in range(NSEL):
            m = jnp.max(cur, axis=-1, keepdims=True)
            idx = jnp.min(jnp.where(cur == m, cols, NBLK),
                          axis=-1, keepdims=True)
            sels.append(idx)
            cur = jnp.where(cols == idx, -1.0, cur)
        t = i * QT + jax.lax.broadcasted_iota(jnp.int32, (QT, T), 0)
        s = jax.lax.broadcasted_iota(jnp.int32, (QT, T), 1)
        sblk = s // SBS
        mask = sblk == (t // SBS)
        for n in range(NSEL):
            mask = mask | (sblk == sels[n])
        mask = mask & (s <= t)
        bias_ref[i] = jnp.where(mask, 0.0, NEG)

    q = q_ref[0]
    k = k_ref[0]
    sim = _ntd(q, k) * SCALE + bias_ref[i]
    m = jnp.max(sim, axis=-1, keepdims=True)
    p = jnp.exp(sim - m)
    denom = jnp.sum(p, axis=-1, keepdims=True)
    fout_ref[0] = _nnd(p, v_ref[0]) / denom


def _window_kernel(q_ref, kp_ref, kc_ref, vp_ref, vc_ref, sout_ref, b_ref):
    i = pl.program_id(0)
    h = pl.program_id(1)

    @pl.when((i == 0) & (h == 0))
    def _():
        r = jax.lax.broadcasted_iota(jnp.int32, (WT, WT), 0)
        c = jax.lax.broadcasted_iota(jnp.int32, (WT, WT), 1)
        dp = r - c + WT
        b_ref[0] = jnp.where((dp >= 0) & (dp < WINDOW), 0.0, NEG)
        dc = r - c
        b_ref[1] = jnp.where((dc >= 0) & (dc < WINDOW), 0.0, NEG)

    q = q_ref[0]
    edge = jnp.where(i == 0, NEG, 0.0)
    sp = _ntd(q, kp_ref[0]) * SCALE + b_ref[0] + edge
    sc = _ntd(q, kc_ref[0]) * SCALE + b_ref[1]
    m = jnp.maximum(jnp.max(sp, axis=-1, keepdims=True),
                    jnp.max(sc, axis=-1, keepdims=True))
    pp = jnp.exp(sp - m)
    pc = jnp.exp(sc - m)
    denom = (jnp.sum(pp, axis=-1, keepdims=True)
             + jnp.sum(pc, axis=-1, keepdims=True))
    sout_ref[0] = (_nnd(pp, vp_ref[0]) + _nnd(pc, vc_ref[0])) / denom


def _combine_kernel(x_ref, co_ref, fo_ref, so_ref,
                    sw_ref, sb_ref, cwh_ref, out_ref):
    xt = x_ref[...]
    glog = _ntd(xt, sw_ref[...]) + sb_ref[...]
    g = 1.0 / (1.0 + jnp.exp(-glog))
    acc = jnp.zeros((QT, DIM), jnp.float32)
    for h in range(HEADS):
        oh = (g[:, 3 * h:3 * h + 1] * co_ref[h]
              + g[:, 3 * h + 1:3 * h + 2] * fo_ref[h]
              + g[:, 3 * h + 2:3 * h + 3] * so_ref[h])
        acc = acc + _ntd(oh, cwh_ref[h])
    out_ref[...] = acc


def _rope_tables():
    inv = 1.0 / (10000.0 ** (jnp.arange(0, DHEAD, 2, dtype=jnp.float32) / DHEAD))
    freqs = jnp.arange(T, dtype=jnp.float32)[:, None] * inv[None, :]
    cos = jnp.repeat(jnp.cos(freqs), 2, axis=1)
    sin = jnp.repeat(jnp.sin(freqs), 2, axis=1)
    return jnp.tile(cos, (1, HEADS)), jnp.tile(sin, (1, HEADS))


@jax.jit
def kernel(x, qkv_w, k_fc_w, k_proj_w, v_fc_w, v_proj_w, compress_mem_kv,
           k_pos, v_pos, strat_w, strat_b, combine_w):
    f32 = jnp.float32
    x2 = x[0]
    wq = qkv_w.reshape(3 * HDIM, DIM)
    cos, sin = _rope_tables()

    qh, kh, vh = pl.pallas_call(
        _qkv_kernel,
        grid=(NQT,),
        in_specs=[
            pl.BlockSpec((QT, DIM), lambda i: (i, 0)),
            pl.BlockSpec((3 * HDIM, DIM), lambda i: (0, 0)),
            pl.BlockSpec((QT, HDIM), lambda i: (i, 0)),
            pl.BlockSpec((QT, HDIM), lambda i: (i, 0)),
        ],
        out_specs=[
            pl.BlockSpec((HEADS, QT, DHEAD), lambda i: (0, i, 0)),
            pl.BlockSpec((HEADS, QT, DHEAD), lambda i: (0, i, 0)),
            pl.BlockSpec((HEADS, QT, DHEAD), lambda i: (0, i, 0)),
        ],
        out_shape=[jax.ShapeDtypeStruct((HEADS, T, DHEAD), f32)] * 3,
    )(x2, wq, cos, sin)

    km = kh.reshape(HEADS, NBLK, CDIM)
    vm = vh.reshape(HEADS, NBLK, CDIM)
    kp = k_pos.reshape(HEADS, 1, CDIM)
    vp = v_pos.reshape(HEADS, 1, CDIM)

    ck, cv = pl.pallas_call(
        _compress_kernel,
        grid=(HEADS,),
        in_specs=[
            pl.BlockSpec((1, NBLK, CDIM), lambda h: (h, 0, 0)),
            pl.BlockSpec((1, NBLK, CDIM), lambda h: (h, 0, 0)),
            pl.BlockSpec((1, 1, CDIM), lambda h: (h, 0, 0)),
            pl.BlockSpec((1, 1, CDIM), lambda h: (h, 0, 0)),
            pl.BlockSpec((HID, CDIM), lambda h: (0, 0)),
            pl.BlockSpec((DHEAD, HID), lambda h: (0, 0)),
            pl.BlockSpec((HID, CDIM), lambda h: (0, 0)),
            pl.BlockSpec((DHEAD, HID), lambda h: (0, 0)),
        ],
        out_specs=[
            pl.BlockSpec((1, NBLK, DHEAD), lambda h: (h, 0, 0)),
            pl.BlockSpec((1, NBLK, DHEAD), lambda h: (h, 0, 0)),
        ],
        out_shape=[jax.ShapeDtypeStruct((HEADS, NBLK, DHEAD), f32)] * 2,
    )(km, vm, kp, vp, k_fc_w, k_proj_w, v_fc_w, v_proj_w)

    mem_k = compress_mem_kv[0].reshape(HEADS, NMEM, DHEAD)
    mem_v = compress_mem_kv[1].reshape(HEADS, NMEM, DHEAD)

    cout, imp = pl.pallas_call(
        _cattn_kernel,
        grid=(NQT, HEADS),
        in_specs=[
            pl.BlockSpec((1, QT, DHEAD), lambda i, h: (h, i, 0)),
            pl.BlockSpec((1, NBLK, DHEAD), lambda i, h: (h, 0, 0)),
            pl.BlockSpec((1, NBLK, DHEAD), lambda i, h: (h, 0, 0)),
            pl.BlockSpec((1, NMEM, DHEAD), lambda i, h: (h, 0, 0)),
            pl.BlockSpec((1, NMEM, DHEAD), lambda i, h: (h, 0, 0)),
        ],
        out_specs=[
            pl.BlockSpec((1, QT, DHEAD), lambda i, h: (h, i, 0)),
            pl.BlockSpec((QT, NBLK), lambda i, h: (i, 0)),
        ],
        out_shape=[
            jax.ShapeDtypeStruct((HEADS, T, DHEAD), f32),
            jax.ShapeDtypeStruct((T, NBLK), f32),
        ],
        scratch_shapes=[pltpu.VMEM((QT, NBLK), f32)],
    )(qh, ck, cv, mem_k, mem_v)

    fout = pl.pallas_call(
        _fine_kernel,
        grid=(HEADS, NQT),
        in_specs=[
            pl.BlockSpec((QT, NBLK),
                         lambda h, i: (jnp.where(h == 0, i, 0), 0)),
            pl.BlockSpec((1, QT, DHEAD), lambda h, i: (h, i, 0)),
            pl.BlockSpec((1, T, DHEAD), lambda h, i: (h, 0, 0)),
            pl.BlockSpec((1, T, DHEAD), lambda h, i: (h, 0, 0)),
        ],
        out_specs=pl.BlockSpec((1, QT, DHEAD), lambda h, i: (h, i, 0)),
        out_shape=jax.ShapeDtypeStruct((HEADS, T, DHEAD), f32),
        scratch_shapes=[pltpu.VMEM((NQT, QT, T), f32)],
    )(imp, qh, kh, vh)

    sout = pl.pallas_call(
        _window_kernel,
        grid=(NWT, HEADS),
        in_specs=[
            pl.BlockSpec((1, WT, DHEAD), lambda i, h: (h, i, 0)),
            pl.BlockSpec((1, WT, DHEAD),
                         lambda i, h: (h, jnp.maximum(i - 1, 0), 0)),
            pl.BlockSpec((1, WT, DHEAD), lambda i, h: (h, i, 0)),
            pl.BlockSpec((1, WT, DHEAD),
                         lambda i, h: (h, jnp.maximum(i - 1, 0), 0)),
            pl.BlockSpec((1, WT, DHEAD), lambda i, h: (h, i, 0)),
        ],
        out_specs=pl.BlockSpec((1, WT, DHEAD), lambda i, h: (h, i, 0)),
        out_shape=jax.ShapeDtypeStruct((HEADS, T, DHEAD), f32),
        scratch_shapes=[pltpu.VMEM((2, WT, WT), f32)],
    )(qh, kh, kh, vh, vh)

    sw = jnp.zeros((128, DIM), f32).at[:3 * HEADS].set(strat_w)
    sb = jnp.zeros((1, 128), f32).at[0, :3 * HEADS].set(strat_b)
    cwh = combine_w.reshape(DIM, HEADS, DHEAD).transpose(1, 0, 2)
    out = pl.pallas_call(
        _combine_kernel,
        grid=(NQT,),
        in_specs=[
            pl.BlockSpec((QT, DIM), lambda i: (i, 0)),
            pl.BlockSpec((HEADS, QT, DHEAD), lambda i: (0, i, 0)),
            pl.BlockSpec((HEADS, QT, DHEAD), lambda i: (0, i, 0)),
            pl.BlockSpec((HEADS, QT, DHEAD), lambda i: (0, i, 0)),
            pl.BlockSpec((128, DIM), lambda i: (0, 0)),
            pl.BlockSpec((1, 128), lambda i: (0, 0)),
            pl.BlockSpec((HEADS, DIM, DHEAD), lambda i: (0, 0, 0)),
        ],
        out_specs=pl.BlockSpec((QT, DIM), lambda i: (i, 0)),
        out_shape=jax.ShapeDtypeStruct((T, DIM), f32),
    )(x2, cout, fout, sout, sw, sb, cwh)

    return out[None]


# speedup vs baseline: 1.6412x; 1.0649x over previous
import functools
import numpy as np
import jax
import jax.numpy as jnp
from jax.experimental import pallas as pl
from jax.experimental.pallas import tpu as pltpu

B, T, DIM = 1, 2048, 768
HEADS, DHEAD = 12, 64
HDIM = HEADS * DHEAD
CBS, SBS = 4, 4
NSEL, NMEM = 4, 1
WINDOW = 32
SCALE = 0.12
CDIM = CBS * DHEAD
HID = CDIM * 4
NBLK = T // CBS

QT = 256
NQT = T // QT
WT = 128
NWT = T // WT

NEG = -1e30


def _nt(a, b):
    return jax.lax.dot_general(a, b, (((1,), (1,)), ((), ())),
                               preferred_element_type=jnp.float32,
                               precision=jax.lax.Precision.HIGHEST)


def _nn(a, b):
    return jax.lax.dot_general(a, b, (((1,), (0,)), ((), ())),
                               preferred_element_type=jnp.float32,
                               precision=jax.lax.Precision.HIGHEST)


def _b(a):
    return a.astype(jnp.bfloat16)


def _ntd(a, b):
    return jax.lax.dot_general(_b(a), _b(b), (((1,), (1,)), ((), ())),
                               preferred_element_type=jnp.float32)


def _nnd(a, b):
    return jax.lax.dot_general(_b(a), _b(b), (((1,), (0,)), ((), ())),
                               preferred_element_type=jnp.float32)


def _rope_rot(x):
    n = x.shape[1]
    zl = pltpu.roll(x, n - 1, 1)
    zr = pltpu.roll(x, 1, 1)
    even = (jax.lax.broadcasted_iota(jnp.int32, x.shape, 1) % 2) == 0
    return jnp.where(even, -zl, zr)


def _qkv_kernel(x_ref, w_ref, c_ref, s_ref, q_ref, k_ref, v_ref):
    xt = x_ref[...]
    qkv = _ntd(xt, w_ref[...])
    q = qkv[:, :HDIM]
    k = qkv[:, HDIM:2 * HDIM]
    v = qkv[:, 2 * HDIM:]
    c = c_ref[...]
    s = s_ref[...]
    qr = q * c + _rope_rot(q) * s
    kr = k * c + _rope_rot(k) * s
    for h in range(HEADS):
        q_ref[h] = qr[:, h * DHEAD:(h + 1) * DHEAD]
        k_ref[h] = kr[:, h * DHEAD:(h + 1) * DHEAD]
        v_ref[h] = v[:, h * DHEAD:(h + 1) * DHEAD]


def _compress_kernel(km_ref, vm_ref, kp_ref, vp_ref,
                     kfc_ref, kpj_ref, vfc_ref, vpj_ref,
                     ck_ref, cv_ref):
    km = km_ref[0] + kp_ref[0]
    hk = _ntd(km, kfc_ref[...])
    hk = jnp.square(jnp.maximum(hk, 0.0))
    ck_ref[0] = _ntd(hk, kpj_ref[...])
    vm = vm_ref[0] + vp_ref[0]
    hv = _ntd(vm, vfc_ref[...])
    hv = jnp.square(jnp.maximum(hv, 0.0))
    cv_ref[0] = _ntd(hv, vpj_ref[...])


def _cattn_kernel(q_ref, ck_ref, cv_ref, mk_ref, mv_ref, cout_ref, imp_ref,
                  bias_ref):
    i = pl.program_id(0)
    h = pl.program_id(1)

    @pl.when(h == 0)
    def _():
        t = i * QT + jax.lax.broadcasted_iota(jnp.int32, (QT, NBLK), 0)
        b = jax.lax.broadcasted_iota(jnp.int32, (QT, NBLK), 1)
        mask = (CBS * b + CBS - 1) < t
        bias_ref[...] = jnp.where(mask, 1.0, 0.0)

    q = q_ref[0]
    ck = ck_ref[0]
    sim = _ntd(q, ck) * SCALE
    qb = _b(q).astype(jnp.float32)
    mkb = _b(mk_ref[0]).astype(jnp.float32)
    mem_logit = jnp.sum(qb * mkb, axis=-1, keepdims=True) * SCALE
    p = jnp.exp(sim) * bias_ref[...]
    pm = jnp.exp(mem_logit)
    denom = jnp.sum(p, axis=-1, keepdims=True) + pm
    attn_n = p / denom
    pm_n = pm / denom
    memo = (_b(pm_n).astype(jnp.float32)) * (_b(mv_ref[0]).astype(jnp.float32))
    cout_ref[0] = _nnd(attn_n, cv_ref[0]) + memo

    @pl.when(h == 0)
    def _():
        imp_ref[...] = attn_n

    @pl.when(h > 0)
    def _():
        imp_ref[...] += attn_n


def _fine_kernel(imp_ref, q_ref, k_ref, v_ref, fout_ref, bias_ref):
    h = pl.program_id(0)
    i = pl.program_id(1)

    @pl.when(h == 0)
    def _():
        imp = imp_ref[...]
        cols = jax.lax.broadcasted_iota(jnp.int32, (QT, NBLK), 1)
        cur = imp
        sels = []
        for n in range(NSEL):
            m = jnp.max(cur, axis=-1, keepdims=True)
            idx = jnp.min(jnp.where(cur == m, cols, NBLK),
                          axis=-1, keepdims=True)
            sels.append(idx)
            cur = jnp.where(cols == idx, -1.0, cur)
        t = i * QT + jax.lax.broadcasted_iota(jnp.int32, (QT, T), 0)
        s = jax.lax.broadcasted_iota(jnp.int32, (QT, T), 1)
        sblk = s // SBS
        mask = sblk == (t // SBS)
        for n in range(NSEL):
            mask = mask | (sblk == sels[n])
        mask = mask & (s <= t)
        bias_ref[i] = jnp.where(mask, 1.0, 0.0)

    q = q_ref[0]
    k = k_ref[0]
    sim = _ntd(q, k) * SCALE
    p = jnp.exp(sim) * bias_ref[i]
    denom = jnp.sum(p, axis=-1, keepdims=True)
    fout_ref[0] = _nnd(p, v_ref[0]) / denom


def _window_kernel(q_ref, kp_ref, kc_ref, vp_ref, vc_ref, sout_ref, b_ref):
    i = pl.program_id(0)
    h = pl.program_id(1)

    @pl.when((i == 0) & (h == 0))
    def _():
        r = jax.lax.broadcasted_iota(jnp.int32, (WT, WT), 0)
        c = jax.lax.broadcasted_iota(jnp.int32, (WT, WT), 1)
        dp = r - c + WT
        b_ref[0] = jnp.where((dp >= 0) & (dp < WINDOW), 1.0, 0.0)
        dc = r - c
        b_ref[1] = jnp.where((dc >= 0) & (dc < WINDOW), 1.0, 0.0)

    q = q_ref[0]
    edge = jnp.where(i == 0, 0.0, 1.0)
    pp = jnp.exp(_ntd(q, kp_ref[0]) * SCALE) * (b_ref[0] * edge)
    pc = jnp.exp(_ntd(q, kc_ref[0]) * SCALE) * b_ref[1]
    denom = (jnp.sum(pp, axis=-1, keepdims=True)
             + jnp.sum(pc, axis=-1, keepdims=True))
    sout_ref[0] = (_nnd(pp, vp_ref[0]) + _nnd(pc, vc_ref[0])) / denom


def _combine_kernel(x_ref, co_ref, fo_ref, so_ref,
                    sw_ref, sb_ref, cwh_ref, out_ref):
    xt = x_ref[...]
    glog = _ntd(xt, sw_ref[...]) + sb_ref[...]
    g = 1.0 / (1.0 + jnp.exp(-glog))
    acc = jnp.zeros((QT, DIM), jnp.float32)
    for h in range(HEADS):
        oh = (g[:, 3 * h:3 * h + 1] * co_ref[h]
              + g[:, 3 * h + 1:3 * h + 2] * fo_ref[h]
              + g[:, 3 * h + 2:3 * h + 3] * so_ref[h])
        acc = acc + _ntd(oh, cwh_ref[h])
    out_ref[...] = acc


def _rope_tables():
    inv = 1.0 / (10000.0 ** (jnp.arange(0, DHEAD, 2, dtype=jnp.float32) / DHEAD))
    freqs = jnp.arange(T, dtype=jnp.float32)[:, None] * inv[None, :]
    cos = jnp.repeat(jnp.cos(freqs), 2, axis=1)
    sin = jnp.repeat(jnp.sin(freqs), 2, axis=1)
    return jnp.tile(cos, (1, HEADS)), jnp.tile(sin, (1, HEADS))


@jax.jit
def kernel(x, qkv_w, k_fc_w, k_proj_w, v_fc_w, v_proj_w, compress_mem_kv,
           k_pos, v_pos, strat_w, strat_b, combine_w):
    f32 = jnp.float32
    x2 = x[0]
    wq = qkv_w.reshape(3 * HDIM, DIM)
    cos, sin = _rope_tables()

    qh, kh, vh = pl.pallas_call(
        _qkv_kernel,
        grid=(NQT,),
        in_specs=[
            pl.BlockSpec((QT, DIM), lambda i: (i, 0)),
            pl.BlockSpec((3 * HDIM, DIM), lambda i: (0, 0)),
            pl.BlockSpec((QT, HDIM), lambda i: (i, 0)),
            pl.BlockSpec((QT, HDIM), lambda i: (i, 0)),
        ],
        out_specs=[
            pl.BlockSpec((HEADS, QT, DHEAD), lambda i: (0, i, 0)),
            pl.BlockSpec((HEADS, QT, DHEAD), lambda i: (0, i, 0)),
            pl.BlockSpec((HEADS, QT, DHEAD), lambda i: (0, i, 0)),
        ],
        out_shape=[jax.ShapeDtypeStruct((HEADS, T, DHEAD), f32)] * 3,
    )(x2, wq, cos, sin)

    km = kh.reshape(HEADS, NBLK, CDIM)
    vm = vh.reshape(HEADS, NBLK, CDIM)
    kp = k_pos.reshape(HEADS, 1, CDIM)
    vp = v_pos.reshape(HEADS, 1, CDIM)

    ck, cv = pl.pallas_call(
        _compress_kernel,
        grid=(HEADS,),
        in_specs=[
            pl.BlockSpec((1, NBLK, CDIM), lambda h: (h, 0, 0)),
            pl.BlockSpec((1, NBLK, CDIM), lambda h: (h, 0, 0)),
            pl.BlockSpec((1, 1, CDIM), lambda h: (h, 0, 0)),
            pl.BlockSpec((1, 1, CDIM), lambda h: (h, 0, 0)),
            pl.BlockSpec((HID, CDIM), lambda h: (0, 0)),
            pl.BlockSpec((DHEAD, HID), lambda h: (0, 0)),
            pl.BlockSpec((HID, CDIM), lambda h: (0, 0)),
            pl.BlockSpec((DHEAD, HID), lambda h: (0, 0)),
        ],
        out_specs=[
            pl.BlockSpec((1, NBLK, DHEAD), lambda h: (h, 0, 0)),
            pl.BlockSpec((1, NBLK, DHEAD), lambda h: (h, 0, 0)),
        ],
        out_shape=[jax.ShapeDtypeStruct((HEADS, NBLK, DHEAD), f32)] * 2,
    )(km, vm, kp, vp, k_fc_w, k_proj_w, v_fc_w, v_proj_w)

    mem_k = compress_mem_kv[0].reshape(HEADS, NMEM, DHEAD)
    mem_v = compress_mem_kv[1].reshape(HEADS, NMEM, DHEAD)

    cout, imp = pl.pallas_call(
        _cattn_kernel,
        grid=(NQT, HEADS),
        in_specs=[
            pl.BlockSpec((1, QT, DHEAD), lambda i, h: (h, i, 0)),
            pl.BlockSpec((1, NBLK, DHEAD), lambda i, h: (h, 0, 0)),
            pl.BlockSpec((1, NBLK, DHEAD), lambda i, h: (h, 0, 0)),
            pl.BlockSpec((1, NMEM, DHEAD), lambda i, h: (h, 0, 0)),
            pl.BlockSpec((1, NMEM, DHEAD), lambda i, h: (h, 0, 0)),
        ],
        out_specs=[
            pl.BlockSpec((1, QT, DHEAD), lambda i, h: (h, i, 0)),
            pl.BlockSpec((QT, NBLK), lambda i, h: (i, 0)),
        ],
        out_shape=[
            jax.ShapeDtypeStruct((HEADS, T, DHEAD), f32),
            jax.ShapeDtypeStruct((T, NBLK), f32),
        ],
        scratch_shapes=[pltpu.VMEM((QT, NBLK), f32)],
    )(qh, ck, cv, mem_k, mem_v)

    fout = pl.pallas_call(
        _fine_kernel,
        grid=(HEADS, NQT),
        in_specs=[
            pl.BlockSpec((QT, NBLK),
                         lambda h, i: (jnp.where(h == 0, i, 0), 0)),
            pl.BlockSpec((1, QT, DHEAD), lambda h, i: (h, i, 0)),
            pl.BlockSpec((1, T, DHEAD), lambda h, i: (h, 0, 0)),
            pl.BlockSpec((1, T, DHEAD), lambda h, i: (h, 0, 0)),
        ],
        out_specs=pl.BlockSpec((1, QT, DHEAD), lambda h, i: (h, i, 0)),
        out_shape=jax.ShapeDtypeStruct((HEADS, T, DHEAD), f32),
        scratch_shapes=[pltpu.VMEM((NQT, QT, T), f32)],
    )(imp, qh, kh, vh)

    sout = pl.pallas_call(
        _window_kernel,
        grid=(NWT, HEADS),
        in_specs=[
            pl.BlockSpec((1, WT, DHEAD), lambda i, h: (h, i, 0)),
            pl.BlockSpec((1, WT, DHEAD),
                         lambda i, h: (h, jnp.maximum(i - 1, 0), 0)),
            pl.BlockSpec((1, WT, DHEAD), lambda i, h: (h, i, 0)),
            pl.BlockSpec((1, WT, DHEAD),
                         lambda i, h: (h, jnp.maximum(i - 1, 0), 0)),
            pl.BlockSpec((1, WT, DHEAD), lambda i, h: (h, i, 0)),
        ],
        out_specs=pl.BlockSpec((1, WT, DHEAD), lambda i, h: (h, i, 0)),
        out_shape=jax.ShapeDtypeStruct((HEADS, T, DHEAD), f32),
        scratch_shapes=[pltpu.VMEM((2, WT, WT), f32)],
    )(qh, kh, kh, vh, vh)

    sw = jnp.zeros((128, DIM), f32).at[:3 * HEADS].set(strat_w)
    sb = jnp.zeros((1, 128), f32).at[0, :3 * HEADS].set(strat_b)
    cwh = combine_w.reshape(DIM, HEADS, DHEAD).transpose(1, 0, 2)
    out = pl.pallas_call(
        _combine_kernel,
        grid=(NQT,),
        in_specs=[
            pl.BlockSpec((QT, DIM), lambda i: (i, 0)),
            pl.BlockSpec((HEADS, QT, DHEAD), lambda i: (0, i, 0)),
            pl.BlockSpec((HEADS, QT, DHEAD), lambda i: (0, i, 0)),
            pl.BlockSpec((HEADS, QT, DHEAD), lambda i: (0, i, 0)),
            pl.BlockSpec((128, DIM), lambda i: (0, 0)),
            pl.BlockSpec((1, 128), lambda i: (0, 0)),
            pl.BlockSpec((HEADS, DIM, DHEAD), lambda i: (0, 0, 0)),
        ],
        out_specs=pl.BlockSpec((QT, DIM), lambda i: (i, 0)),
        out_shape=jax.ShapeDtypeStruct((T, DIM), f32),
    )(x2, cout, fout, sout, sw, sb, cwh)

    return out[None]


# speedup vs baseline: 2.1427x; 1.3056x over previous
import functools
import numpy as np
import jax
import jax.numpy as jnp
from jax.experimental import pallas as pl
from jax.experimental.pallas import tpu as pltpu

B, T, DIM = 1, 2048, 768
HEADS, DHEAD = 12, 64
HDIM = HEADS * DHEAD
CBS, SBS = 4, 4
NSEL, NMEM = 4, 1
WINDOW = 32
SCALE = 0.12
CDIM = CBS * DHEAD
HID = CDIM * 4
NBLK = T // CBS

QT = 256
NQT = T // QT
WT = 128
NWT = T // WT

NEG = -1e30


def _nt(a, b):
    return jax.lax.dot_general(a, b, (((1,), (1,)), ((), ())),
                               preferred_element_type=jnp.float32,
                               precision=jax.lax.Precision.HIGHEST)


def _nn(a, b):
    return jax.lax.dot_general(a, b, (((1,), (0,)), ((), ())),
                               preferred_element_type=jnp.float32,
                               precision=jax.lax.Precision.HIGHEST)


def _b(a):
    return a.astype(jnp.bfloat16)


def _ntd(a, b):
    return jax.lax.dot_general(_b(a), _b(b), (((1,), (1,)), ((), ())),
                               preferred_element_type=jnp.float32)


def _nnd(a, b):
    return jax.lax.dot_general(_b(a), _b(b), (((1,), (0,)), ((), ())),
                               preferred_element_type=jnp.float32)


def _rope_rot(x):
    n = x.shape[1]
    zl = pltpu.roll(x, n - 1, 1)
    zr = pltpu.roll(x, 1, 1)
    even = (jax.lax.broadcasted_iota(jnp.int32, x.shape, 1) % 2) == 0
    return jnp.where(even, -zl, zr)


def _qkv_kernel(x_ref, w_ref, c_ref, s_ref, q_ref, k_ref, v_ref):
    xt = x_ref[...]
    qkv = _ntd(xt, w_ref[...])
    q = qkv[:, :HDIM]
    k = qkv[:, HDIM:2 * HDIM]
    v = qkv[:, 2 * HDIM:]
    c = c_ref[...]
    s = s_ref[...]
    qr = q * c + _rope_rot(q) * s
    kr = k * c + _rope_rot(k) * s
    for h in range(HEADS):
        q_ref[h] = qr[:, h * DHEAD:(h + 1) * DHEAD]
        k_ref[h] = kr[:, h * DHEAD:(h + 1) * DHEAD]
        v_ref[h] = v[:, h * DHEAD:(h + 1) * DHEAD]


def _compress_kernel(km_ref, vm_ref, kp_ref, vp_ref,
                     kfc_ref, kpj_ref, vfc_ref, vpj_ref,
                     ck_ref, cv_ref):
    km = km_ref[0] + kp_ref[0]
    hk = _ntd(km, kfc_ref[...])
    hk = jnp.square(jnp.maximum(hk, 0.0))
    ck_ref[0] = _ntd(hk, kpj_ref[...])
    vm = vm_ref[0] + vp_ref[0]
    hv = _ntd(vm, vfc_ref[...])
    hv = jnp.square(jnp.maximum(hv, 0.0))
    cv_ref[0] = _ntd(hv, vpj_ref[...])


def _cattn_kernel(q_ref, ck_ref, cv_ref, mk_ref, mv_ref, cout_ref, imp_ref,
                  bias_ref):
    i = pl.program_id(0)
    h = pl.program_id(1)

    @pl.when(h == 0)
    def _():
        t = i * QT + jax.lax.broadcasted_iota(jnp.int32, (QT, NBLK), 0)
        b = jax.lax.broadcasted_iota(jnp.int32, (QT, NBLK), 1)
        mask = (CBS * b + CBS - 1) < t
        bias_ref[...] = jnp.where(mask, 1.0, 0.0)

    q = q_ref[0]
    ck = ck_ref[0]
    sim = _ntd(q, ck) * SCALE
    qb = _b(q).astype(jnp.float32)
    mkb = _b(mk_ref[0]).astype(jnp.float32)
    mem_logit = jnp.sum(qb * mkb, axis=-1, keepdims=True) * SCALE
    p = jnp.exp(sim) * bias_ref[...]
    pm = jnp.exp(mem_logit)
    denom = jnp.sum(p, axis=-1, keepdims=True) + pm
    attn_n = p / denom
    pm_n = pm / denom
    memo = (_b(pm_n).astype(jnp.float32)) * (_b(mv_ref[0]).astype(jnp.float32))
    cout_ref[0] = _nnd(attn_n, cv_ref[0]) + memo

    @pl.when(h == 0)
    def _():
        imp_ref[...] = attn_n

    @pl.when(h > 0)
    def _():
        imp_ref[...] += attn_n


WKEY = QT + 128


def _fine_kernel(imp_ref, q_ref, k_ref, v_ref, fout_ref, sout_ref,
                 bias_ref, wm_ref):
    h = pl.program_id(0)
    i = pl.program_id(1)

    @pl.when((h == 0) & (i == 0))
    def _():
        r = jax.lax.broadcasted_iota(jnp.int32, (QT, WKEY), 0)
        c = jax.lax.broadcasted_iota(jnp.int32, (QT, WKEY), 1)
        d0 = r - c
        wm_ref[0] = jnp.where((d0 >= 0) & (d0 < WINDOW), 1.0, 0.0)
        d1 = r - c + 128
        wm_ref[1] = jnp.where((d1 >= 0) & (d1 < WINDOW), 1.0, 0.0)

    @pl.when(h == 0)
    def _():
        imp = imp_ref[...]
        cols = jax.lax.broadcasted_iota(jnp.int32, (QT, NBLK), 1)
        cur = imp
        sels = []
        for n in range(NSEL):
            m = jnp.max(cur, axis=-1, keepdims=True)
            idx = jnp.min(jnp.where(cur == m, cols, NBLK),
                          axis=-1, keepdims=True)
            sels.append(idx)
            cur = jnp.where(cols == idx, -1.0, cur)
        t = i * QT + jax.lax.broadcasted_iota(jnp.int32, (QT, T), 0)
        s = jax.lax.broadcasted_iota(jnp.int32, (QT, T), 1)
        sblk = s // SBS
        mask = sblk == (t // SBS)
        for n in range(NSEL):
            mask = mask | (sblk == sels[n])
        mask = mask & (s <= t)
        bias_ref[i] = jnp.where(mask, 1.0, 0.0)

    q = q_ref[0]
    k = k_ref[0]
    sim = _ntd(q, k) * SCALE
    p = jnp.exp(sim) * bias_ref[i]
    denom = jnp.sum(p, axis=-1, keepdims=True)
    fout_ref[0] = _nnd(p, v_ref[0]) / denom

    base = jnp.maximum(i * QT - 128, 0)
    ks = k_ref[0, pl.ds(base, WKEY), :]
    vs = v_ref[0, pl.ds(base, WKEY), :]
    pw = jnp.exp(_ntd(q, ks) * SCALE) * wm_ref[jnp.minimum(i, 1)]
    denw = jnp.sum(pw, axis=-1, keepdims=True)
    sout_ref[0] = _nnd(pw, vs) / denw


def _combine_kernel(x_ref, co_ref, fo_ref, so_ref,
                    sw_ref, sb_ref, cwh_ref, out_ref):
    xt = x_ref[...]
    glog = _ntd(xt, sw_ref[...]) + sb_ref[...]
    g = 1.0 / (1.0 + jnp.exp(-glog))
    acc = jnp.zeros((QT, DIM), jnp.float32)
    for h in range(HEADS):
        oh = (g[:, 3 * h:3 * h + 1] * co_ref[h]
              + g[:, 3 * h + 1:3 * h + 2] * fo_ref[h]
              + g[:, 3 * h + 2:3 * h + 3] * so_ref[h])
        acc = acc + _ntd(oh, cwh_ref[h])
    out_ref[...] = acc


def _rope_tables():
    inv = 1.0 / (10000.0 ** (jnp.arange(0, DHEAD, 2, dtype=jnp.float32) / DHEAD))
    freqs = jnp.arange(T, dtype=jnp.float32)[:, None] * inv[None, :]
    cos = jnp.repeat(jnp.cos(freqs), 2, axis=1)
    sin = jnp.repeat(jnp.sin(freqs), 2, axis=1)
    return jnp.tile(cos, (1, HEADS)), jnp.tile(sin, (1, HEADS))


@jax.jit
def kernel(x, qkv_w, k_fc_w, k_proj_w, v_fc_w, v_proj_w, compress_mem_kv,
           k_pos, v_pos, strat_w, strat_b, combine_w):
    f32 = jnp.float32
    x2 = x[0]
    wq = qkv_w.reshape(3 * HDIM, DIM)
    cos, sin = _rope_tables()

    qh, kh, vh = pl.pallas_call(
        _qkv_kernel,
        grid=(NQT,),
        in_specs=[
            pl.BlockSpec((QT, DIM), lambda i: (i, 0)),
            pl.BlockSpec((3 * HDIM, DIM), lambda i: (0, 0)),
            pl.BlockSpec((QT, HDIM), lambda i: (i, 0)),
            pl.BlockSpec((QT, HDIM), lambda i: (i, 0)),
        ],
        out_specs=[
            pl.BlockSpec((HEADS, QT, DHEAD), lambda i: (0, i, 0)),
            pl.BlockSpec((HEADS, QT, DHEAD), lambda i: (0, i, 0)),
            pl.BlockSpec((HEADS, QT, DHEAD), lambda i: (0, i, 0)),
        ],
        out_shape=[jax.ShapeDtypeStruct((HEADS, T, DHEAD), f32)] * 3,
    )(x2, wq, cos, sin)

    km = kh.reshape(HEADS, NBLK, CDIM)
    vm = vh.reshape(HEADS, NBLK, CDIM)
    kp = k_pos.reshape(HEADS, 1, CDIM)
    vp = v_pos.reshape(HEADS, 1, CDIM)

    ck, cv = pl.pallas_call(
        _compress_kernel,
        grid=(HEADS,),
        in_specs=[
            pl.BlockSpec((1, NBLK, CDIM), lambda h: (h, 0, 0)),
            pl.BlockSpec((1, NBLK, CDIM), lambda h: (h, 0, 0)),
            pl.BlockSpec((1, 1, CDIM), lambda h: (h, 0, 0)),
            pl.BlockSpec((1, 1, CDIM), lambda h: (h, 0, 0)),
            pl.BlockSpec((HID, CDIM), lambda h: (0, 0)),
            pl.BlockSpec((DHEAD, HID), lambda h: (0, 0)),
            pl.BlockSpec((HID, CDIM), lambda h: (0, 0)),
            pl.BlockSpec((DHEAD, HID), lambda h: (0, 0)),
        ],
        out_specs=[
            pl.BlockSpec((1, NBLK, DHEAD), lambda h: (h, 0, 0)),
            pl.BlockSpec((1, NBLK, DHEAD), lambda h: (h, 0, 0)),
        ],
        out_shape=[jax.ShapeDtypeStruct((HEADS, NBLK, DHEAD), f32)] * 2,
    )(km, vm, kp, vp, k_fc_w, k_proj_w, v_fc_w, v_proj_w)

    mem_k = compress_mem_kv[0].reshape(HEADS, NMEM, DHEAD)
    mem_v = compress_mem_kv[1].reshape(HEADS, NMEM, DHEAD)

    cout, imp = pl.pallas_call(
        _cattn_kernel,
        grid=(NQT, HEADS),
        in_specs=[
            pl.BlockSpec((1, QT, DHEAD), lambda i, h: (h, i, 0)),
            pl.BlockSpec((1, NBLK, DHEAD), lambda i, h: (h, 0, 0)),
            pl.BlockSpec((1, NBLK, DHEAD), lambda i, h: (h, 0, 0)),
            pl.BlockSpec((1, NMEM, DHEAD), lambda i, h: (h, 0, 0)),
            pl.BlockSpec((1, NMEM, DHEAD), lambda i, h: (h, 0, 0)),
        ],
        out_specs=[
            pl.BlockSpec((1, QT, DHEAD), lambda i, h: (h, i, 0)),
            pl.BlockSpec((QT, NBLK), lambda i, h: (i, 0)),
        ],
        out_shape=[
            jax.ShapeDtypeStruct((HEADS, T, DHEAD), f32),
            jax.ShapeDtypeStruct((T, NBLK), f32),
        ],
        scratch_shapes=[pltpu.VMEM((QT, NBLK), f32)],
    )(qh, ck, cv, mem_k, mem_v)

    fout, sout = pl.pallas_call(
        _fine_kernel,
        grid=(HEADS, NQT),
        in_specs=[
            pl.BlockSpec((QT, NBLK),
                         lambda h, i: (jnp.where(h == 0, i, 0), 0)),
            pl.BlockSpec((1, QT, DHEAD), lambda h, i: (h, i, 0)),
            pl.BlockSpec((1, T, DHEAD), lambda h, i: (h, 0, 0)),
            pl.BlockSpec((1, T, DHEAD), lambda h, i: (h, 0, 0)),
        ],
        out_specs=[
            pl.BlockSpec((1, QT, DHEAD), lambda h, i: (h, i, 0)),
            pl.BlockSpec((1, QT, DHEAD), lambda h, i: (h, i, 0)),
        ],
        out_shape=[jax.ShapeDtypeStruct((HEADS, T, DHEAD), f32)] * 2,
        scratch_shapes=[pltpu.VMEM((NQT, QT, T), f32),
                        pltpu.VMEM((2, QT, WKEY), f32)],
    )(imp, qh, kh, vh)

    sw = jnp.zeros((128, DIM), f32).at[:3 * HEADS].set(strat_w)
    sb = jnp.zeros((1, 128), f32).at[0, :3 * HEADS].set(strat_b)
    cwh = combine_w.reshape(DIM, HEADS, DHEAD).transpose(1, 0, 2)
    out = pl.pallas_call(
        _combine_kernel,
        grid=(NQT,),
        in_specs=[
            pl.BlockSpec((QT, DIM), lambda i: (i, 0)),
            pl.BlockSpec((HEADS, QT, DHEAD), lambda i: (0, i, 0)),
            pl.BlockSpec((HEADS, QT, DHEAD), lambda i: (0, i, 0)),
            pl.BlockSpec((HEADS, QT, DHEAD), lambda i: (0, i, 0)),
            pl.BlockSpec((128, DIM), lambda i: (0, 0)),
            pl.BlockSpec((1, 128), lambda i: (0, 0)),
            pl.BlockSpec((HEADS, DIM, DHEAD), lambda i: (0, 0, 0)),
        ],
        out_specs=pl.BlockSpec((QT, DIM), lambda i: (i, 0)),
        out_shape=jax.ShapeDtypeStruct((T, DIM), f32),
    )(x2, cout, fout, sout, sw, sb, cwh)

    return out[None]


# speedup vs baseline: 2.1952x; 1.0245x over previous
import functools
import numpy as np
import jax
import jax.numpy as jnp
from jax.experimental import pallas as pl
from jax.experimental.pallas import tpu as pltpu

B, T, DIM = 1, 2048, 768
HEADS, DHEAD = 12, 64
HDIM = HEADS * DHEAD
CBS, SBS = 4, 4
NSEL, NMEM = 4, 1
WINDOW = 32
SCALE = 0.12
CDIM = CBS * DHEAD
HID = CDIM * 4
NBLK = T // CBS

QT = 256
NQT = T // QT
WT = 128
NWT = T // WT

NEG = -1e30


def _nt(a, b):
    return jax.lax.dot_general(a, b, (((1,), (1,)), ((), ())),
                               preferred_element_type=jnp.float32,
                               precision=jax.lax.Precision.HIGHEST)


def _nn(a, b):
    return jax.lax.dot_general(a, b, (((1,), (0,)), ((), ())),
                               preferred_element_type=jnp.float32,
                               precision=jax.lax.Precision.HIGHEST)


def _b(a):
    return a.astype(jnp.bfloat16)


def _ntd(a, b):
    return jax.lax.dot_general(_b(a), _b(b), (((1,), (1,)), ((), ())),
                               preferred_element_type=jnp.float32)


def _nnd(a, b):
    return jax.lax.dot_general(_b(a), _b(b), (((1,), (0,)), ((), ())),
                               preferred_element_type=jnp.float32)


def _rope_rot(x):
    n = x.shape[1]
    zl = pltpu.roll(x, n - 1, 1)
    zr = pltpu.roll(x, 1, 1)
    even = (jax.lax.broadcasted_iota(jnp.int32, x.shape, 1) % 2) == 0
    return jnp.where(even, -zl, zr)


def _qkv_kernel(x_ref, w_ref, c_ref, s_ref, q_ref, k_ref, v_ref):
    xt = x_ref[...]
    qkv = _ntd(xt, w_ref[...])
    q = qkv[:, :HDIM]
    k = qkv[:, HDIM:2 * HDIM]
    v = qkv[:, 2 * HDIM:]
    c = c_ref[...]
    s = s_ref[...]
    qr = q * c + _rope_rot(q) * s
    kr = k * c + _rope_rot(k) * s
    for h in range(HEADS):
        q_ref[h] = qr[:, h * DHEAD:(h + 1) * DHEAD]
        k_ref[h] = kr[:, h * DHEAD:(h + 1) * DHEAD]
        v_ref[h] = v[:, h * DHEAD:(h + 1) * DHEAD]


def _compress_kernel(km_ref, vm_ref, kp_ref, vp_ref,
                     kfc_ref, kpj_ref, vfc_ref, vpj_ref,
                     ck_ref, cv_ref):
    km = km_ref[0] + kp_ref[0]
    hk = _ntd(km, kfc_ref[...])
    hk = jnp.square(jnp.maximum(hk, 0.0))
    ck_ref[0] = _ntd(hk, kpj_ref[...])
    vm = vm_ref[0] + vp_ref[0]
    hv = _ntd(vm, vfc_ref[...])
    hv = jnp.square(jnp.maximum(hv, 0.0))
    cv_ref[0] = _ntd(hv, vpj_ref[...])


def _cattn_kernel(q_ref, ck_ref, cv_ref, mk_ref, mv_ref, cout_ref, imp_ref,
                  bias_ref, acc_ref):
    h = pl.program_id(0)
    i = pl.program_id(1)

    @pl.when(h == 0)
    def _():
        t = i * QT + jax.lax.broadcasted_iota(jnp.int32, (QT, NBLK), 0)
        b = jax.lax.broadcasted_iota(jnp.int32, (QT, NBLK), 1)
        mask = (CBS * b + CBS - 1) < t
        bias_ref[i] = jnp.where(mask, 1.0, 0.0)

    q = q_ref[0]
    ck = ck_ref[0]
    sim = _ntd(q, ck) * SCALE
    qb = _b(q).astype(jnp.float32)
    mkb = _b(mk_ref[0]).astype(jnp.float32)
    mem_logit = jnp.sum(qb * mkb, axis=-1, keepdims=True) * SCALE
    p = jnp.exp(sim) * bias_ref[i]
    pm = jnp.exp(mem_logit)
    denom = jnp.sum(p, axis=-1, keepdims=True) + pm
    attn_n = p / denom
    pm_n = pm / denom
    memo = (_b(pm_n).astype(jnp.float32)) * (_b(mv_ref[0]).astype(jnp.float32))
    cout_ref[0] = _nnd(attn_n, cv_ref[0]) + memo

    @pl.when(h == 0)
    def _():
        acc_ref[i] = attn_n

    @pl.when(h > 0)
    def _():
        acc_ref[i] += attn_n

    @pl.when((h == HEADS - 1) & (i == NQT - 1))
    def _():
        for n in range(NQT):
            imp_ref[n * QT:(n + 1) * QT] = acc_ref[n]


WKEY = QT + 128


def _fine_kernel(imp_ref, q_ref, k_ref, v_ref, fout_ref, sout_ref,
                 bias_ref, wm_ref):
    h = pl.program_id(0)
    i = pl.program_id(1)

    @pl.when((h == 0) & (i == 0))
    def _():
        r = jax.lax.broadcasted_iota(jnp.int32, (QT, WKEY), 0)
        c = jax.lax.broadcasted_iota(jnp.int32, (QT, WKEY), 1)
        d0 = r - c
        wm_ref[0] = jnp.where((d0 >= 0) & (d0 < WINDOW), 1.0, 0.0)
        d1 = r - c + 128
        wm_ref[1] = jnp.where((d1 >= 0) & (d1 < WINDOW), 1.0, 0.0)

    @pl.when(h == 0)
    def _():
        imp = imp_ref[...]
        cols = jax.lax.broadcasted_iota(jnp.int32, (QT, NBLK), 1)
        cur = imp
        sels = []
        for n in range(NSEL):
            m = jnp.max(cur, axis=-1, keepdims=True)
            idx = jnp.min(jnp.where(cur == m, cols, NBLK),
                          axis=-1, keepdims=True)
            sels.append(idx)
            cur = jnp.where(cols == idx, -1.0, cur)
        t = i * QT + jax.lax.broadcasted_iota(jnp.int32, (QT, T), 0)
        s = jax.lax.broadcasted_iota(jnp.int32, (QT, T), 1)
        sblk = s // SBS
        mask = sblk == (t // SBS)
        for n in range(NSEL):
            mask = mask | (sblk == sels[n])
        mask = mask & (s <= t)
        bias_ref[i] = jnp.where(mask, 1.0, 0.0)

    q = q_ref[0]
    k = k_ref[0]
    sim = _ntd(q, k) * SCALE
    p = jnp.exp(sim) * bias_ref[i]
    denom = jnp.sum(p, axis=-1, keepdims=True)
    fout_ref[0] = _nnd(p, v_ref[0]) / denom

    base = jnp.maximum(i * QT - 128, 0)
    ks = k_ref[0, pl.ds(base, WKEY), :]
    vs = v_ref[0, pl.ds(base, WKEY), :]
    pw = jnp.exp(_ntd(q, ks) * SCALE) * wm_ref[jnp.minimum(i, 1)]
    denw = jnp.sum(pw, axis=-1, keepdims=True)
    sout_ref[0] = _nnd(pw, vs) / denw


def _combine_kernel(x_ref, co_ref, fo_ref, so_ref,
                    sw_ref, sb_ref, cwh_ref, out_ref):
    xt = x_ref[...]
    glog = _ntd(xt, sw_ref[...]) + sb_ref[...]
    g = 1.0 / (1.0 + jnp.exp(-glog))
    acc = jnp.zeros((QT, DIM), jnp.float32)
    for h in range(HEADS):
        oh = (g[:, 3 * h:3 * h + 1] * co_ref[h]
              + g[:, 3 * h + 1:3 * h + 2] * fo_ref[h]
              + g[:, 3 * h + 2:3 * h + 3] * so_ref[h])
        acc = acc + _ntd(oh, cwh_ref[h])
    out_ref[...] = acc


def _rope_tables():
    inv = 1.0 / (10000.0 ** (jnp.arange(0, DHEAD, 2, dtype=jnp.float32) / DHEAD))
    freqs = jnp.arange(T, dtype=jnp.float32)[:, None] * inv[None, :]
    cos = jnp.repeat(jnp.cos(freqs), 2, axis=1)
    sin = jnp.repeat(jnp.sin(freqs), 2, axis=1)
    return jnp.tile(cos, (1, HEADS)), jnp.tile(sin, (1, HEADS))


@jax.jit
def kernel(x, qkv_w, k_fc_w, k_proj_w, v_fc_w, v_proj_w, compress_mem_kv,
           k_pos, v_pos, strat_w, strat_b, combine_w):
    f32 = jnp.float32
    x2 = x[0]
    wq = qkv_w.reshape(3 * HDIM, DIM)
    cos, sin = _rope_tables()

    qh, kh, vh = pl.pallas_call(
        _qkv_kernel,
        grid=(NQT,),
        in_specs=[
            pl.BlockSpec((QT, DIM), lambda i: (i, 0)),
            pl.BlockSpec((3 * HDIM, DIM), lambda i: (0, 0)),
            pl.BlockSpec((QT, HDIM), lambda i: (i, 0)),
            pl.BlockSpec((QT, HDIM), lambda i: (i, 0)),
        ],
        out_specs=[
            pl.BlockSpec((HEADS, QT, DHEAD), lambda i: (0, i, 0)),
            pl.BlockSpec((HEADS, QT, DHEAD), lambda i: (0, i, 0)),
            pl.BlockSpec((HEADS, QT, DHEAD), lambda i: (0, i, 0)),
        ],
        out_shape=[jax.ShapeDtypeStruct((HEADS, T, DHEAD), f32)] * 3,
    )(x2, wq, cos, sin)

    km = kh.reshape(HEADS, NBLK, CDIM)
    vm = vh.reshape(HEADS, NBLK, CDIM)
    kp = k_pos.reshape(HEADS, 1, CDIM)
    vp = v_pos.reshape(HEADS, 1, CDIM)

    ck, cv = pl.pallas_call(
        _compress_kernel,
        grid=(HEADS,),
        in_specs=[
            pl.BlockSpec((1, NBLK, CDIM), lambda h: (h, 0, 0)),
            pl.BlockSpec((1, NBLK, CDIM), lambda h: (h, 0, 0)),
            pl.BlockSpec((1, 1, CDIM), lambda h: (h, 0, 0)),
            pl.BlockSpec((1, 1, CDIM), lambda h: (h, 0, 0)),
            pl.BlockSpec((HID, CDIM), lambda h: (0, 0)),
            pl.BlockSpec((DHEAD, HID), lambda h: (0, 0)),
            pl.BlockSpec((HID, CDIM), lambda h: (0, 0)),
            pl.BlockSpec((DHEAD, HID), lambda h: (0, 0)),
        ],
        out_specs=[
            pl.BlockSpec((1, NBLK, DHEAD), lambda h: (h, 0, 0)),
            pl.BlockSpec((1, NBLK, DHEAD), lambda h: (h, 0, 0)),
        ],
        out_shape=[jax.ShapeDtypeStruct((HEADS, NBLK, DHEAD), f32)] * 2,
    )(km, vm, kp, vp, k_fc_w, k_proj_w, v_fc_w, v_proj_w)

    mem_k = compress_mem_kv[0].reshape(HEADS, NMEM, DHEAD)
    mem_v = compress_mem_kv[1].reshape(HEADS, NMEM, DHEAD)

    cout, imp = pl.pallas_call(
        _cattn_kernel,
        grid=(HEADS, NQT),
        in_specs=[
            pl.BlockSpec((1, QT, DHEAD), lambda h, i: (h, i, 0)),
            pl.BlockSpec((1, NBLK, DHEAD), lambda h, i: (h, 0, 0)),
            pl.BlockSpec((1, NBLK, DHEAD), lambda h, i: (h, 0, 0)),
            pl.BlockSpec((1, NMEM, DHEAD), lambda h, i: (h, 0, 0)),
            pl.BlockSpec((1, NMEM, DHEAD), lambda h, i: (h, 0, 0)),
        ],
        out_specs=[
            pl.BlockSpec((1, QT, DHEAD), lambda h, i: (h, i, 0)),
            pl.BlockSpec((T, NBLK), lambda h, i: (0, 0)),
        ],
        out_shape=[
            jax.ShapeDtypeStruct((HEADS, T, DHEAD), f32),
            jax.ShapeDtypeStruct((T, NBLK), f32),
        ],
        scratch_shapes=[pltpu.VMEM((NQT, QT, NBLK), f32),
                        pltpu.VMEM((NQT, QT, NBLK), f32)],
    )(qh, ck, cv, mem_k, mem_v)

    fout, sout = pl.pallas_call(
        _fine_kernel,
        grid=(HEADS, NQT),
        in_specs=[
            pl.BlockSpec((QT, NBLK),
                         lambda h, i: (jnp.where(h == 0, i, 0), 0)),
            pl.BlockSpec((1, QT, DHEAD), lambda h, i: (h, i, 0)),
            pl.BlockSpec((1, T, DHEAD), lambda h, i: (h, 0, 0)),
            pl.BlockSpec((1, T, DHEAD), lambda h, i: (h, 0, 0)),
        ],
        out_specs=[
            pl.BlockSpec((1, QT, DHEAD), lambda h, i: (h, i, 0)),
            pl.BlockSpec((1, QT, DHEAD), lambda h, i: (h, i, 0)),
        ],
        out_shape=[jax.ShapeDtypeStruct((HEADS, T, DHEAD), f32)] * 2,
        scratch_shapes=[pltpu.VMEM((NQT, QT, T), f32),
                        pltpu.VMEM((2, QT, WKEY), f32)],
    )(imp, qh, kh, vh)

    sw = jnp.zeros((128, DIM), f32).at[:3 * HEADS].set(strat_w)
    sb = jnp.zeros((1, 128), f32).at[0, :3 * HEADS].set(strat_b)
    cwh = combine_w.reshape(DIM, HEADS, DHEAD).transpose(1, 0, 2)
    out = pl.pallas_call(
        _combine_kernel,
        grid=(NQT,),
        in_specs=[
            pl.BlockSpec((QT, DIM), lambda i: (i, 0)),
            pl.BlockSpec((HEADS, QT, DHEAD), lambda i: (0, i, 0)),
            pl.BlockSpec((HEADS, QT, DHEAD), lambda i: (0, i, 0)),
            pl.BlockSpec((HEADS, QT, DHEAD), lambda i: (0, i, 0)),
            pl.BlockSpec((128, DIM), lambda i: (0, 0)),
            pl.BlockSpec((1, 128), lambda i: (0, 0)),
            pl.BlockSpec((HEADS, DIM, DHEAD), lambda i: (0, 0, 0)),
        ],
        out_specs=pl.BlockSpec((QT, DIM), lambda i: (i, 0)),
        out_shape=jax.ShapeDtypeStruct((T, DIM), f32),
    )(x2, cout, fout, sout, sw, sb, cwh)

    return out[None]


# speedup vs baseline: 2.2716x; 1.0348x over previous
import functools
import numpy as np
import jax
import jax.numpy as jnp
from jax.experimental import pallas as pl
from jax.experimental.pallas import tpu as pltpu

B, T, DIM = 1, 2048, 768
HEADS, DHEAD = 12, 64
HDIM = HEADS * DHEAD
CBS, SBS = 4, 4
NSEL, NMEM = 4, 1
WINDOW = 32
SCALE = 0.12
CDIM = CBS * DHEAD
HID = CDIM * 4
NBLK = T // CBS

QT = 256
NQT = T // QT
WT = 128
NWT = T // WT

NEG = -1e30


def _nt(a, b):
    return jax.lax.dot_general(a, b, (((1,), (1,)), ((), ())),
                               preferred_element_type=jnp.float32,
                               precision=jax.lax.Precision.HIGHEST)


def _nn(a, b):
    return jax.lax.dot_general(a, b, (((1,), (0,)), ((), ())),
                               preferred_element_type=jnp.float32,
                               precision=jax.lax.Precision.HIGHEST)


def _b(a):
    return a.astype(jnp.bfloat16)


def _ntd(a, b):
    return jax.lax.dot_general(_b(a), _b(b), (((1,), (1,)), ((), ())),
                               preferred_element_type=jnp.float32)


def _nnd(a, b):
    return jax.lax.dot_general(_b(a), _b(b), (((1,), (0,)), ((), ())),
                               preferred_element_type=jnp.float32)


def _rope_rot(x):
    n = x.shape[1]
    zl = pltpu.roll(x, n - 1, 1)
    zr = pltpu.roll(x, 1, 1)
    even = (jax.lax.broadcasted_iota(jnp.int32, x.shape, 1) % 2) == 0
    return jnp.where(even, -zl, zr)


def _qkv_kernel(x_ref, w_ref, c_ref, s_ref, q_ref, k_ref, v_ref):
    xt = x_ref[...]
    qkv = _ntd(xt, w_ref[...])
    q = qkv[:, :HDIM]
    k = qkv[:, HDIM:2 * HDIM]
    v = qkv[:, 2 * HDIM:]
    c = c_ref[...]
    s = s_ref[...]
    qr = q * c + _rope_rot(q) * s
    kr = k * c + _rope_rot(k) * s
    for h in range(HEADS):
        q_ref[h] = qr[:, h * DHEAD:(h + 1) * DHEAD]
        k_ref[h] = kr[:, h * DHEAD:(h + 1) * DHEAD]
        v_ref[h] = v[:, h * DHEAD:(h + 1) * DHEAD]


def _compress_kernel(km_ref, vm_ref, kp_ref, vp_ref,
                     kfc_ref, kpj_ref, vfc_ref, vpj_ref,
                     ck_ref, cv_ref):
    km = km_ref[0] + kp_ref[0]
    hk = _ntd(km, kfc_ref[...])
    hk = jnp.square(jnp.maximum(hk, 0.0))
    ck_ref[0] = _ntd(hk, kpj_ref[...])
    vm = vm_ref[0] + vp_ref[0]
    hv = _ntd(vm, vfc_ref[...])
    hv = jnp.square(jnp.maximum(hv, 0.0))
    cv_ref[0] = _ntd(hv, vpj_ref[...])


def _cattn_kernel(q_ref, ck_ref, cv_ref, mk_ref, mv_ref, cout_ref, imp_ref,
                  bias_ref, acc_ref):
    h = pl.program_id(0)
    i = pl.program_id(1)

    @pl.when(h == 0)
    def _():
        t = i * QT + jax.lax.broadcasted_iota(jnp.int32, (QT, NBLK), 0)
        b = jax.lax.broadcasted_iota(jnp.int32, (QT, NBLK), 1)
        mask = (CBS * b + CBS - 1) < t
        bias_ref[i] = jnp.where(mask, 1.0, 0.0)

    q = q_ref[0]
    ck = ck_ref[0]
    sim = _ntd(q, ck) * SCALE
    qb = _b(q).astype(jnp.float32)
    mkb = _b(mk_ref[0]).astype(jnp.float32)
    mem_logit = jnp.sum(qb * mkb, axis=-1, keepdims=True) * SCALE
    p = jnp.exp(sim) * bias_ref[i]
    pm = jnp.exp(mem_logit)
    denom = jnp.sum(p, axis=-1, keepdims=True) + pm
    attn_n = p / denom
    pm_n = pm / denom
    memo = (_b(pm_n).astype(jnp.float32)) * (_b(mv_ref[0]).astype(jnp.float32))
    cout_ref[0] = _nnd(attn_n, cv_ref[0]) + memo

    @pl.when(h == 0)
    def _():
        acc_ref[i] = attn_n

    @pl.when(h > 0)
    def _():
        acc_ref[i] += attn_n

    @pl.when((h == HEADS - 1) & (i == NQT - 1))
    def _():
        for n in range(NQT):
            imp_ref[n * QT:(n + 1) * QT] = acc_ref[n]


WKEY = QT + 128


def _fine_kernel(imp_ref, q_ref, k_ref, v_ref, fout_ref, sout_ref,
                 bias_ref, wm_ref):
    h = pl.program_id(0)
    i = pl.program_id(1)

    @pl.when((h == 0) & (i == 0))
    def _():
        r = jax.lax.broadcasted_iota(jnp.int32, (QT, WKEY), 0)
        c = jax.lax.broadcasted_iota(jnp.int32, (QT, WKEY), 1)
        d0 = r - c
        wm_ref[0] = jnp.where((d0 >= 0) & (d0 < WINDOW), 1.0, 0.0)
        d1 = r - c + 128
        wm_ref[1] = jnp.where((d1 >= 0) & (d1 < WINDOW), 1.0, 0.0)

    @pl.when(h == 0)
    def _():
        imp = imp_ref[...]
        cols = jax.lax.broadcasted_iota(jnp.int32, (QT, NBLK), 1)
        cur = imp
        sels = []
        for n in range(NSEL):
            m = jnp.max(cur, axis=-1, keepdims=True)
            idx = jnp.min(jnp.where(cur == m, cols, NBLK),
                          axis=-1, keepdims=True)
            sels.append(idx)
            cur = jnp.where(cols == idx, -1.0, cur)
        t = i * QT + jax.lax.broadcasted_iota(jnp.int32, (QT, T), 0)
        s = jax.lax.broadcasted_iota(jnp.int32, (QT, T), 1)
        sblk = s // SBS
        mask = sblk == (t // SBS)
        for n in range(NSEL):
            mask = mask | (sblk == sels[n])
        mask = mask & (s <= t)
        bias_ref[i] = jnp.where(mask, 1.0, 0.0)

    q = q_ref[0]

    for c in range(4):
        @pl.when(i // 2 == c)
        def _(c=c):
            w = 2 * QT * (c + 1)
            k = k_ref[0, :w, :]
            sim = _ntd(q, k) * SCALE
            p = jnp.exp(sim) * bias_ref[i, :, :w]
            denom = jnp.sum(p, axis=-1, keepdims=True)
            fout_ref[0] = _nnd(p, v_ref[0, :w, :]) / denom

    base = jnp.maximum(i * QT - 128, 0)
    ks = k_ref[0, pl.ds(base, WKEY), :]
    vs = v_ref[0, pl.ds(base, WKEY), :]
    pw = jnp.exp(_ntd(q, ks) * SCALE) * wm_ref[jnp.minimum(i, 1)]
    denw = jnp.sum(pw, axis=-1, keepdims=True)
    sout_ref[0] = _nnd(pw, vs) / denw


def _combine_kernel(x_ref, co_ref, fo_ref, so_ref,
                    sw_ref, sb_ref, cwh_ref, out_ref):
    xt = x_ref[...]
    glog = _ntd(xt, sw_ref[...]) + sb_ref[...]
    g = 1.0 / (1.0 + jnp.exp(-glog))
    acc = jnp.zeros((QT, DIM), jnp.float32)
    for h in range(HEADS):
        oh = (g[:, 3 * h:3 * h + 1] * co_ref[h]
              + g[:, 3 * h + 1:3 * h + 2] * fo_ref[h]
              + g[:, 3 * h + 2:3 * h + 3] * so_ref[h])
        acc = acc + _ntd(oh, cwh_ref[h])
    out_ref[...] = acc


def _rope_tables():
    inv = 1.0 / (10000.0 ** (jnp.arange(0, DHEAD, 2, dtype=jnp.float32) / DHEAD))
    freqs = jnp.arange(T, dtype=jnp.float32)[:, None] * inv[None, :]
    cos = jnp.repeat(jnp.cos(freqs), 2, axis=1)
    sin = jnp.repeat(jnp.sin(freqs), 2, axis=1)
    return jnp.tile(cos, (1, HEADS)), jnp.tile(sin, (1, HEADS))


@jax.jit
def kernel(x, qkv_w, k_fc_w, k_proj_w, v_fc_w, v_proj_w, compress_mem_kv,
           k_pos, v_pos, strat_w, strat_b, combine_w):
    f32 = jnp.float32
    x2 = x[0]
    wq = qkv_w.reshape(3 * HDIM, DIM)
    cos, sin = _rope_tables()

    qh, kh, vh = pl.pallas_call(
        _qkv_kernel,
        grid=(NQT,),
        in_specs=[
            pl.BlockSpec((QT, DIM), lambda i: (i, 0)),
            pl.BlockSpec((3 * HDIM, DIM), lambda i: (0, 0)),
            pl.BlockSpec((QT, HDIM), lambda i: (i, 0)),
            pl.BlockSpec((QT, HDIM), lambda i: (i, 0)),
        ],
        out_specs=[
            pl.BlockSpec((HEADS, QT, DHEAD), lambda i: (0, i, 0)),
            pl.BlockSpec((HEADS, QT, DHEAD), lambda i: (0, i, 0)),
            pl.BlockSpec((HEADS, QT, DHEAD), lambda i: (0, i, 0)),
        ],
        out_shape=[jax.ShapeDtypeStruct((HEADS, T, DHEAD), f32)] * 3,
    )(x2, wq, cos, sin)

    km = kh.reshape(HEADS, NBLK, CDIM)
    vm = vh.reshape(HEADS, NBLK, CDIM)
    kp = k_pos.reshape(HEADS, 1, CDIM)
    vp = v_pos.reshape(HEADS, 1, CDIM)

    ck, cv = pl.pallas_call(
        _compress_kernel,
        grid=(HEADS,),
        in_specs=[
            pl.BlockSpec((1, NBLK, CDIM), lambda h: (h, 0, 0)),
            pl.BlockSpec((1, NBLK, CDIM), lambda h: (h, 0, 0)),
            pl.BlockSpec((1, 1, CDIM), lambda h: (h, 0, 0)),
            pl.BlockSpec((1, 1, CDIM), lambda h: (h, 0, 0)),
            pl.BlockSpec((HID, CDIM), lambda h: (0, 0)),
            pl.BlockSpec((DHEAD, HID), lambda h: (0, 0)),
            pl.BlockSpec((HID, CDIM), lambda h: (0, 0)),
            pl.BlockSpec((DHEAD, HID), lambda h: (0, 0)),
        ],
        out_specs=[
            pl.BlockSpec((1, NBLK, DHEAD), lambda h: (h, 0, 0)),
            pl.BlockSpec((1, NBLK, DHEAD), lambda h: (h, 0, 0)),
        ],
        out_shape=[jax.ShapeDtypeStruct((HEADS, NBLK, DHEAD), f32)] * 2,
    )(km, vm, kp, vp, k_fc_w, k_proj_w, v_fc_w, v_proj_w)

    mem_k = compress_mem_kv[0].reshape(HEADS, NMEM, DHEAD)
    mem_v = compress_mem_kv[1].reshape(HEADS, NMEM, DHEAD)

    cout, imp = pl.pallas_call(
        _cattn_kernel,
        grid=(HEADS, NQT),
        in_specs=[
            pl.BlockSpec((1, QT, DHEAD), lambda h, i: (h, i, 0)),
            pl.BlockSpec((1, NBLK, DHEAD), lambda h, i: (h, 0, 0)),
            pl.BlockSpec((1, NBLK, DHEAD), lambda h, i: (h, 0, 0)),
            pl.BlockSpec((1, NMEM, DHEAD), lambda h, i: (h, 0, 0)),
            pl.BlockSpec((1, NMEM, DHEAD), lambda h, i: (h, 0, 0)),
        ],
        out_specs=[
            pl.BlockSpec((1, QT, DHEAD), lambda h, i: (h, i, 0)),
            pl.BlockSpec((T, NBLK), lambda h, i: (0, 0)),
        ],
        out_shape=[
            jax.ShapeDtypeStruct((HEADS, T, DHEAD), f32),
            jax.ShapeDtypeStruct((T, NBLK), f32),
        ],
        scratch_shapes=[pltpu.VMEM((NQT, QT, NBLK), f32),
                        pltpu.VMEM((NQT, QT, NBLK), f32)],
    )(qh, ck, cv, mem_k, mem_v)

    fout, sout = pl.pallas_call(
        _fine_kernel,
        grid=(HEADS, NQT),
        in_specs=[
            pl.BlockSpec((QT, NBLK),
                         lambda h, i: (jnp.where(h == 0, i, 0), 0)),
            pl.BlockSpec((1, QT, DHEAD), lambda h, i: (h, i, 0)),
            pl.BlockSpec((1, T, DHEAD), lambda h, i: (h, 0, 0)),
            pl.BlockSpec((1, T, DHEAD), lambda h, i: (h, 0, 0)),
        ],
        out_specs=[
            pl.BlockSpec((1, QT, DHEAD), lambda h, i: (h, i, 0)),
            pl.BlockSpec((1, QT, DHEAD), lambda h, i: (h, i, 0)),
        ],
        out_shape=[jax.ShapeDtypeStruct((HEADS, T, DHEAD), f32)] * 2,
        scratch_shapes=[pltpu.VMEM((NQT, QT, T), f32),
                        pltpu.VMEM((2, QT, WKEY), f32)],
    )(imp, qh, kh, vh)

    sw = jnp.zeros((128, DIM), f32).at[:3 * HEADS].set(strat_w)
    sb = jnp.zeros((1, 128), f32).at[0, :3 * HEADS].set(strat_b)
    cwh = combine_w.reshape(DIM, HEADS, DHEAD).transpose(1, 0, 2)
    out = pl.pallas_call(
        _combine_kernel,
        grid=(NQT,),
        in_specs=[
            pl.BlockSpec((QT, DIM), lambda i: (i, 0)),
            pl.BlockSpec((HEADS, QT, DHEAD), lambda i: (0, i, 0)),
            pl.BlockSpec((HEADS, QT, DHEAD), lambda i: (0, i, 0)),
            pl.BlockSpec((HEADS, QT, DHEAD), lambda i: (0, i, 0)),
            pl.BlockSpec((128, DIM), lambda i: (0, 0)),
            pl.BlockSpec((1, 128), lambda i: (0, 0)),
            pl.BlockSpec((HEADS, DIM, DHEAD), lambda i: (0, 0, 0)),
        ],
        out_specs=pl.BlockSpec((QT, DIM), lambda i: (i, 0)),
        out_shape=jax.ShapeDtypeStruct((T, DIM), f32),
    )(x2, cout, fout, sout, sw, sb, cwh)

    return out[None]
